# baseline (speedup 1.0000x reference)
import functools

import jax
import jax.numpy as jnp
import numpy as np
from jax import lax
from jax.experimental import pallas as pl
from jax.experimental.pallas import tpu as pltpu
from jax.experimental.pallas import tpu_sc as plsc

_NPOINT = 512
_NSAMPLE = 32
_RADIUS2 = np.float32(0.2 ** 2)
_TS = 128
_NW = 32
_CH = 128


def _fps_body(xt_ref, out_ref):
    B = xt_ref.shape[0]
    N = xt_ref.shape[2]
    R = N // 128
    x = xt_ref[:, 0, :].reshape(B, R, 128)
    y = xt_ref[:, 1, :].reshape(B, R, 128)
    z = xt_ref[:, 2, :].reshape(B, R, 128)
    iota = (lax.broadcasted_iota(jnp.int32, (B, R, 128), 1) * 128
            + lax.broadcasted_iota(jnp.int32, (B, R, 128), 2))
    lane = lax.broadcasted_iota(jnp.int32, (1, 1, 128), 2)
    oh0 = (lane == 0).astype(jnp.float32)
    oh1 = (lane == 1).astype(jnp.float32)
    oh2 = (lane == 2).astype(jnp.float32)

    def bred(op, a):
        return op(op(a, axis=2), axis=1)

    def gather3(nxt):
        one = iota == nxt[:, None, None]
        gx = bred(jnp.sum, jnp.where(one, x, 0.0))
        gy = bred(jnp.sum, jnp.where(one, y, 0.0))
        gz = bred(jnp.sum, jnp.where(one, z, 0.0))
        return gx, gy, gz

    dist0 = jnp.full((B, R, 128), 1e10, jnp.float32)
    cx0, cy0, cz0 = gather3(jnp.zeros((B,), jnp.int32))

    def body(i, carry):
        dist, cx, cy, cz = carry
        rows = (cx[:, None, None] * oh0 + cy[:, None, None] * oh1
                + cz[:, None, None] * oh2)
        out_ref[:, pl.ds(i - 1, 1), :] = rows
        dx = x - cx[:, None, None]
        dy = y - cy[:, None, None]
        dz = z - cz[:, None, None]
        d = dx * dx + dy * dy
        d = d + dz * dz
        dist = jnp.minimum(dist, d)
        m = bred(jnp.max, dist)
        cand = jnp.where(dist == m[:, None, None], iota, N)
        nxt = bred(jnp.min, cand)
        gx, gy, gz = gather3(nxt)
        return dist, gx, gy, gz

    _, cx, cy, cz = lax.fori_loop(1, _NPOINT, body, (dist0, cx0, cy0, cz0))
    rows = (cx[:, None, None] * oh0 + cy[:, None, None] * oh1
            + cz[:, None, None] * oh2)
    out_ref[:, pl.ds(_NPOINT - 1, 1), :] = rows


def _fps(xt):
    B, _, N = xt.shape
    return pl.pallas_call(
        _fps_body,
        out_shape=jax.ShapeDtypeStruct((B, _NPOINT, 128), jnp.float32),
    )(xt)


def _bq_body(xt_ref, nt_ref, out_ref):
    N = xt_ref.shape[2]
    x = xt_ref[0, 0, :]
    y = xt_ref[0, 1, :]
    z = xt_ref[0, 2, :]
    cx = nt_ref[0, 0, :]
    cy = nt_ref[0, 1, :]
    cz = nt_ref[0, 2, :]
    dx = cx[:, None] - x[None, :]
    dy = cy[:, None] - y[None, :]
    dz = cz[:, None] - z[None, :]
    d2 = dx * dx + dy * dy
    d2 = d2 + dz * dz
    iota = lax.broadcasted_iota(jnp.int32, (_TS, N), 1)
    cand = jnp.where(d2 < _RADIUS2, iota, N)
    first = jnp.min(cand, axis=1)
    cols = []
    cur = cand
    for _ in range(_NSAMPLE):
        v = jnp.min(cur, axis=1)
        cols.append(jnp.where(v == N, first, v)[:, None])
        cur = jnp.where(cur == v[:, None], N, cur)
    out_ref[0] = jnp.concatenate(cols, axis=1)


def _ball_query(xt, nt):
    B, _, N = xt.shape
    S = nt.shape[2]
    return pl.pallas_call(
        _bq_body,
        grid=(B, S // _TS),
        in_specs=[
            pl.BlockSpec((1, 3, N), lambda b, t: (b, 0, 0)),
            pl.BlockSpec((1, 3, _TS), lambda b, t: (b, 0, t)),
        ],
        out_specs=pl.BlockSpec((1, _TS, _NSAMPLE), lambda b, t: (b, t, 0)),
        out_shape=jax.ShapeDtypeStruct((B, S, _NSAMPLE), jnp.int32),
    )(xt, nt)


def _sc_gather(tbl, idxg):
    total = idxg.shape[0]
    d = tbl.shape[1]
    per_w = total // _NW
    n_ch = per_w // _CH
    mesh = plsc.VectorSubcoreMesh(core_axis_name="c", subcore_axis_name="s")

    @functools.partial(
        pl.kernel,
        mesh=mesh,
        out_type=jax.ShapeDtypeStruct((total, d), jnp.float32),
        scratch_types=[
            pltpu.VMEM((_CH,), jnp.int32),
            pltpu.VMEM((_CH, d), jnp.float32),
            pltpu.SemaphoreType.DMA,
        ],
    )
    def gk(tbl_hbm, idx_hbm, out_hbm, idx_v, rows_v, sem):
        wid = lax.axis_index("s") * 2 + lax.axis_index("c")
        base = wid * per_w

        def step(j, carry):
            off = base + j * _CH
            pltpu.sync_copy(idx_hbm.at[pl.ds(off, _CH)], idx_v)
            pltpu.async_copy(tbl_hbm.at[idx_v], rows_v, sem).wait()
            pltpu.sync_copy(rows_v, out_hbm.at[pl.ds(off, _CH)])
            return carry

        lax.fori_loop(0, n_ch, step, 0)

    return gk(tbl, idxg)


def _mlp_body(g_ref, nx_ref, w1_ref, w1c_ref, b1_ref, w2_ref, b2_ref,
              w3_ref, b3_ref, out_ref):
    K = _NSAMPLE
    g = g_ref[0]
    h = jnp.dot(g, w1_ref[...], preferred_element_type=jnp.float32)
    c = nx_ref[0]
    ct = jnp.dot(c, w1c_ref[...], preferred_element_type=jnp.float32)
    h = h.reshape(_TS, K, h.shape[-1]) - ct[:, None, :]
    h = jnp.maximum(h + b1_ref[...][None], 0.0)
    h = h.reshape(_TS * K, h.shape[-1])
    h = jnp.maximum(
        jnp.dot(h, w2_ref[...], preferred_element_type=jnp.float32)
        + b2_ref[...], 0.0)
    h = jnp.maximum(
        jnp.dot(h, w3_ref[...], preferred_element_type=jnp.float32)
        + b3_ref[...], 0.0)
    out_ref[0] = jnp.max(h.reshape(_TS, K, h.shape[-1]), axis=1)


def _mlp(g, nxp, wbig, w1c, b1, w2, b2, w3, b3):
    B = g.shape[0]
    S = nxp.shape[1]
    D = g.shape[2]
    C3 = w3.shape[1]
    full = lambda shp: pl.BlockSpec(shp, lambda b, t: tuple(0 for _ in shp))
    return pl.pallas_call(
        _mlp_body,
        grid=(B, S // _TS),
        in_specs=[
            pl.BlockSpec((1, _TS * _NSAMPLE, D), lambda b, t: (b, t, 0)),
            pl.BlockSpec((1, _TS, 8), lambda b, t: (b, t, 0)),
            full(wbig.shape),
            full(w1c.shape),
            full(b1.shape),
            full(w2.shape),
            full(b2.shape),
            full(w3.shape),
            full(b3.shape),
        ],
        out_specs=pl.BlockSpec((1, _TS, C3), lambda b, t: (b, t, 0)),
        out_shape=jax.ShapeDtypeStruct((B, S, C3), jnp.float32),
    )(g, nxp, wbig, w1c, b1, w2, b2, w3, b3)


def kernel(xyz, features, W1, b1, W2, b2, W3, b3):
    B, N, _ = xyz.shape
    C = features.shape[1]
    S, K = _NPOINT, _NSAMPLE
    f32 = jnp.float32

    xt = jnp.transpose(xyz, (0, 2, 1))
    nx_pad = _fps(xt)
    new_xyz = nx_pad[:, :, :3]
    nt = jnp.transpose(new_xyz, (0, 2, 1))
    idx = _ball_query(xt, nt)
    return (new_xyz, idx.astype(jnp.float32))

    pad = (-(C + 3)) % 128
    D = C + 3 + pad
    feats_t = jnp.transpose(features, (0, 2, 1))
    tbl = jnp.concatenate(
        [feats_t, xyz, jnp.zeros((B, N, pad), f32)], axis=-1
    ).reshape(B * N, D)
    idxg = (idx + (jnp.arange(B, dtype=jnp.int32) * N)[:, None, None]
            ).reshape(-1)
    g = _sc_gather(tbl, idxg).reshape(B, S * K, D)

    nxp = jnp.concatenate([new_xyz, jnp.zeros((B, S, 5), f32)], axis=-1)
    wbig = jnp.concatenate(
        [W1[3:], W1[:3], jnp.zeros((pad, W1.shape[1]), f32)], axis=0)
    w1c = jnp.concatenate([W1[:3], jnp.zeros((5, W1.shape[1]), f32)], axis=0)
    out = _mlp(g, nxp, wbig, w1c, b1.reshape(1, -1), W2, b2.reshape(1, -1),
               W3, b3.reshape(1, -1))
    new_features = jnp.transpose(out, (0, 2, 1))
    return (new_xyz, new_features)


# speedup vs baseline: 1.4192x; 1.4192x over previous
import functools

import jax
import jax.numpy as jnp
import numpy as np
from jax import lax
from jax.experimental import pallas as pl
from jax.experimental.pallas import tpu as pltpu
from jax.experimental.pallas import tpu_sc as plsc

_NPOINT = 512
_NSAMPLE = 32
_RADIUS2 = np.float32(0.2 ** 2)
_TS = 128
_NW = 32
_CH = 128


def _fps_body(xt_ref, out_ref):
    B = xt_ref.shape[0]
    N = xt_ref.shape[2]
    R = N // 128
    xs, ys, zs, xall = [], [], [], []
    for b in range(B):
        x = xt_ref[b, 0, :].reshape(R, 128)
        y = xt_ref[b, 1, :].reshape(R, 128)
        z = xt_ref[b, 2, :].reshape(R, 128)
        xs.append(x); ys.append(y); zs.append(z)
        xall.append(jnp.concatenate([x, y, z], axis=1))
    iota = (lax.broadcasted_iota(jnp.int32, (R, 128), 0) * 128
            + lax.broadcasted_iota(jnp.int32, (R, 128), 1))
    lane = lax.broadcasted_iota(jnp.int32, (1, 128), 1)
    rowi = lax.broadcasted_iota(jnp.int32, (1, R), 1)
    oh0 = (lane == 0).astype(jnp.float32)
    oh1 = (lane == 1).astype(jnp.float32)
    oh2 = (lane == 2).astype(jnp.float32)

    def gather3(nxt, b):
        r = lax.shift_right_logical(nxt, 7)
        l = jnp.bitwise_and(nxt, 127)
        oneR = (rowi == r).astype(jnp.float32)
        oneL = (lane == l).astype(jnp.float32)
        tmp = jnp.dot(oneR, xall[b], preferred_element_type=jnp.float32)
        cx = jnp.sum(tmp[:, :128] * oneL, axis=1, keepdims=True)
        cy = jnp.sum(tmp[:, 128:256] * oneL, axis=1, keepdims=True)
        cz = jnp.sum(tmp[:, 256:] * oneL, axis=1, keepdims=True)
        return cx, cy, cz

    dist0, c0 = [], []
    for b in range(B):
        dist0.append(jnp.full((R, 128), 1e10, jnp.float32))
        c0.append((xs[b][0:1, 0:1], ys[b][0:1, 0:1], zs[b][0:1, 0:1]))

    def body(i, carry):
        new = []
        for b in range(B):
            dist, (cx, cy, cz) = carry[b]
            row = cx * oh0 + cy * oh1 + cz * oh2
            out_ref[b, pl.ds(i - 1, 1), :] = row
            dx = xs[b] - cx
            dy = ys[b] - cy
            dz = zs[b] - cz
            d = dx * dx + dy * dy
            d = d + dz * dz
            dist = jnp.minimum(dist, d)
            m = jnp.max(dist, axis=1, keepdims=True)
            m = jnp.max(m, axis=0, keepdims=True)
            cand = jnp.where(dist == m, iota, N)
            nxt = jnp.min(cand, axis=1, keepdims=True)
            nxt = jnp.min(nxt, axis=0, keepdims=True)
            new.append((dist, gather3(nxt, b)))
        return tuple(new)

    carry = lax.fori_loop(1, _NPOINT, body,
                          tuple((dist0[b], c0[b]) for b in range(B)))
    for b in range(B):
        _, (cx, cy, cz) = carry[b]
        row = cx * oh0 + cy * oh1 + cz * oh2
        out_ref[b, pl.ds(_NPOINT - 1, 1), :] = row


def _fps(xt):
    B, _, N = xt.shape
    return pl.pallas_call(
        _fps_body,
        out_shape=jax.ShapeDtypeStruct((B, _NPOINT, 128), jnp.float32),
    )(xt)


def _bq_body(xt_ref, nt_ref, out_ref):
    N = xt_ref.shape[2]
    x = xt_ref[0, 0, :]
    y = xt_ref[0, 1, :]
    z = xt_ref[0, 2, :]
    cx = nt_ref[0, 0, :]
    cy = nt_ref[0, 1, :]
    cz = nt_ref[0, 2, :]
    dx = cx[:, None] - x[None, :]
    dy = cy[:, None] - y[None, :]
    dz = cz[:, None] - z[None, :]
    d2 = dx * dx + dy * dy
    d2 = d2 + dz * dz
    iota = lax.broadcasted_iota(jnp.int32, (_TS, N), 1)
    cand = jnp.where(d2 < _RADIUS2, iota, N)
    first = jnp.min(cand, axis=1)
    cols = []
    cur = cand
    for _ in range(_NSAMPLE):
        v = jnp.min(cur, axis=1)
        cols.append(jnp.where(v == N, first, v)[:, None])
        cur = jnp.where(cur == v[:, None], N, cur)
    out_ref[0] = jnp.concatenate(cols, axis=1)


def _ball_query(xt, nt):
    B, _, N = xt.shape
    S = nt.shape[2]
    return pl.pallas_call(
        _bq_body,
        grid=(B, S // _TS),
        in_specs=[
            pl.BlockSpec((1, 3, N), lambda b, t: (b, 0, 0)),
            pl.BlockSpec((1, 3, _TS), lambda b, t: (b, 0, t)),
        ],
        out_specs=pl.BlockSpec((1, _TS, _NSAMPLE), lambda b, t: (b, t, 0)),
        out_shape=jax.ShapeDtypeStruct((B, S, _NSAMPLE), jnp.int32),
    )(xt, nt)


def _sc_gather(tbl, idxg):
    total = idxg.shape[0]
    d = tbl.shape[1]
    per_w = total // _NW
    n_ch = per_w // _CH
    mesh = plsc.VectorSubcoreMesh(core_axis_name="c", subcore_axis_name="s")

    @functools.partial(
        pl.kernel,
        mesh=mesh,
        out_type=jax.ShapeDtypeStruct((total, d), jnp.float32),
        scratch_types=[
            pltpu.VMEM((_CH,), jnp.int32),
            pltpu.VMEM((_CH, d), jnp.float32),
            pltpu.SemaphoreType.DMA,
        ],
    )
    def gk(tbl_hbm, idx_hbm, out_hbm, idx_v, rows_v, sem):
        wid = lax.axis_index("s") * 2 + lax.axis_index("c")
        base = wid * per_w

        def step(j, carry):
            off = base + j * _CH
            pltpu.sync_copy(idx_hbm.at[pl.ds(off, _CH)], idx_v)
            pltpu.async_copy(tbl_hbm.at[idx_v], rows_v, sem).wait()
            pltpu.sync_copy(rows_v, out_hbm.at[pl.ds(off, _CH)])
            return carry

        lax.fori_loop(0, n_ch, step, 0)

    return gk(tbl, idxg)


def _mlp_body(g_ref, nx_ref, w1_ref, w1c_ref, b1_ref, w2_ref, b2_ref,
              w3_ref, b3_ref, out_ref):
    K = _NSAMPLE
    g = g_ref[0]
    h = jnp.dot(g, w1_ref[...], preferred_element_type=jnp.float32)
    c = nx_ref[0]
    ct = jnp.dot(c, w1c_ref[...], preferred_element_type=jnp.float32)
    h = h.reshape(_TS, K, h.shape[-1]) - ct[:, None, :]
    h = jnp.maximum(h + b1_ref[...][None], 0.0)
    h = h.reshape(_TS * K, h.shape[-1])
    h = jnp.maximum(
        jnp.dot(h, w2_ref[...], preferred_element_type=jnp.float32)
        + b2_ref[...], 0.0)
    h = jnp.maximum(
        jnp.dot(h, w3_ref[...], preferred_element_type=jnp.float32)
        + b3_ref[...], 0.0)
    out_ref[0] = jnp.max(h.reshape(_TS, K, h.shape[-1]), axis=1)


def _mlp(g, nxp, wbig, w1c, b1, w2, b2, w3, b3):
    B = g.shape[0]
    S = nxp.shape[1]
    D = g.shape[2]
    C3 = w3.shape[1]
    full = lambda shp: pl.BlockSpec(shp, lambda b, t: tuple(0 for _ in shp))
    return pl.pallas_call(
        _mlp_body,
        grid=(B, S // _TS),
        in_specs=[
            pl.BlockSpec((1, _TS * _NSAMPLE, D), lambda b, t: (b, t, 0)),
            pl.BlockSpec((1, _TS, 8), lambda b, t: (b, t, 0)),
            full(wbig.shape),
            full(w1c.shape),
            full(b1.shape),
            full(w2.shape),
            full(b2.shape),
            full(w3.shape),
            full(b3.shape),
        ],
        out_specs=pl.BlockSpec((1, _TS, C3), lambda b, t: (b, t, 0)),
        out_shape=jax.ShapeDtypeStruct((B, S, C3), jnp.float32),
    )(g, nxp, wbig, w1c, b1, w2, b2, w3, b3)


def kernel(xyz, features, W1, b1, W2, b2, W3, b3):
    B, N, _ = xyz.shape
    C = features.shape[1]
    S, K = _NPOINT, _NSAMPLE
    f32 = jnp.float32

    xt = jnp.transpose(xyz, (0, 2, 1))
    nx_pad = _fps(xt)
    new_xyz = nx_pad[:, :, :3]
    nt = jnp.transpose(new_xyz, (0, 2, 1))
    idx = _ball_query(xt, nt)

    pad = (-(C + 3)) % 128
    D = C + 3 + pad
    feats_t = jnp.transpose(features, (0, 2, 1))
    tbl = jnp.concatenate(
        [feats_t, xyz, jnp.zeros((B, N, pad), f32)], axis=-1
    ).reshape(B * N, D)
    idxg = (idx + (jnp.arange(B, dtype=jnp.int32) * N)[:, None, None]
            ).reshape(-1)
    g = _sc_gather(tbl, idxg).reshape(B, S * K, D)

    nxp = jnp.concatenate([new_xyz, jnp.zeros((B, S, 5), f32)], axis=-1)
    wbig = jnp.concatenate(
        [W1[3:], W1[:3], jnp.zeros((pad, W1.shape[1]), f32)], axis=0)
    w1c = jnp.concatenate([W1[:3], jnp.zeros((5, W1.shape[1]), f32)], axis=0)
    out = _mlp(g, nxp, wbig, w1c, b1.reshape(1, -1), W2, b2.reshape(1, -1),
               W3, b3.reshape(1, -1))
    new_features = jnp.transpose(out, (0, 2, 1))
    return (new_xyz, new_features)


# speedup vs baseline: 1.5140x; 1.0669x over previous
import functools

import jax
import jax.numpy as jnp
import numpy as np
from jax import lax
from jax.experimental import pallas as pl
from jax.experimental.pallas import tpu as pltpu
from jax.experimental.pallas import tpu_sc as plsc

_NPOINT = 512
_NSAMPLE = 32
_RADIUS2 = np.float32(0.2 ** 2)
_TS = 128
_NW = 32
_CH = 128


def _fps_body(xt_ref, out_ref):
    B = xt_ref.shape[0]
    N = xt_ref.shape[2]
    R = N // 128
    xs, ys, zs, xall = [], [], [], []
    for b in range(B):
        x = xt_ref[b, 0, :].reshape(R, 128)
        y = xt_ref[b, 1, :].reshape(R, 128)
        z = xt_ref[b, 2, :].reshape(R, 128)
        xs.append(x); ys.append(y); zs.append(z)
        xall.append(jnp.concatenate([x, y, z], axis=1))
    iota = (lax.broadcasted_iota(jnp.int32, (R, 128), 0) * 128
            + lax.broadcasted_iota(jnp.int32, (R, 128), 1))
    lane = lax.broadcasted_iota(jnp.int32, (1, 128), 1)
    rowc = lax.broadcasted_iota(jnp.int32, (R, 1), 0)
    oh0 = (lane == 0).astype(jnp.float32)
    oh1 = (lane == 1).astype(jnp.float32)
    oh2 = (lane == 2).astype(jnp.float32)

    def gather3(nxt, b):
        r = lax.shift_right_logical(nxt, 7)
        l = jnp.bitwise_and(nxt, 127)
        oneR = (rowc == r).astype(jnp.float32)
        oneL = (lane == l).astype(jnp.float32)
        tmp = jnp.sum(xall[b] * oneR, axis=0, keepdims=True)
        cx = jnp.sum(tmp[:, :128] * oneL, axis=1, keepdims=True)
        cy = jnp.sum(tmp[:, 128:256] * oneL, axis=1, keepdims=True)
        cz = jnp.sum(tmp[:, 256:] * oneL, axis=1, keepdims=True)
        return cx, cy, cz

    dist0, c0 = [], []
    for b in range(B):
        dist0.append(jnp.full((R, 128), 1e10, jnp.float32))
        c0.append((xs[b][0:1, 0:1], ys[b][0:1, 0:1], zs[b][0:1, 0:1]))

    def body(i, carry):
        new = []
        for b in range(B):
            dist, (cx, cy, cz) = carry[b]
            row = cx * oh0 + cy * oh1 + cz * oh2
            out_ref[b, pl.ds(i - 1, 1), :] = row
            dx = xs[b] - cx
            dy = ys[b] - cy
            dz = zs[b] - cz
            d = dx * dx + dy * dy
            d = d + dz * dz
            dist = jnp.minimum(dist, d)
            m = jnp.max(dist, axis=1, keepdims=True)
            m = jnp.max(m, axis=0, keepdims=True)
            cand = jnp.where(dist == m, iota, N)
            nxt = jnp.min(cand, axis=1, keepdims=True)
            nxt = jnp.min(nxt, axis=0, keepdims=True)
            new.append((dist, gather3(nxt, b)))
        return tuple(new)

    carry = lax.fori_loop(1, _NPOINT, body,
                          tuple((dist0[b], c0[b]) for b in range(B)))
    for b in range(B):
        _, (cx, cy, cz) = carry[b]
        row = cx * oh0 + cy * oh1 + cz * oh2
        out_ref[b, pl.ds(_NPOINT - 1, 1), :] = row


def _fps(xt):
    B, _, N = xt.shape
    return pl.pallas_call(
        _fps_body,
        out_shape=jax.ShapeDtypeStruct((B, _NPOINT, 128), jnp.float32),
    )(xt)


def _bq_body(xt_ref, nt_ref, out_ref):
    N = xt_ref.shape[2]
    x = xt_ref[0, 0, :]
    y = xt_ref[0, 1, :]
    z = xt_ref[0, 2, :]
    cx = nt_ref[0, 0, :]
    cy = nt_ref[0, 1, :]
    cz = nt_ref[0, 2, :]
    dx = cx[:, None] - x[None, :]
    dy = cy[:, None] - y[None, :]
    dz = cz[:, None] - z[None, :]
    d2 = dx * dx + dy * dy
    d2 = d2 + dz * dz
    iota = lax.broadcasted_iota(jnp.int32, (_TS, N), 1)
    cand = jnp.where(d2 < _RADIUS2, iota, N)
    first = jnp.min(cand, axis=1)
    cols = []
    cur = cand
    for _ in range(_NSAMPLE):
        v = jnp.min(cur, axis=1)
        cols.append(jnp.where(v == N, first, v)[:, None])
        cur = jnp.where(cur == v[:, None], N, cur)
    out_ref[0] = jnp.concatenate(cols, axis=1)


def _ball_query(xt, nt):
    B, _, N = xt.shape
    S = nt.shape[2]
    return pl.pallas_call(
        _bq_body,
        grid=(B, S // _TS),
        in_specs=[
            pl.BlockSpec((1, 3, N), lambda b, t: (b, 0, 0)),
            pl.BlockSpec((1, 3, _TS), lambda b, t: (b, 0, t)),
        ],
        out_specs=pl.BlockSpec((1, _TS, _NSAMPLE), lambda b, t: (b, t, 0)),
        out_shape=jax.ShapeDtypeStruct((B, S, _NSAMPLE), jnp.int32),
    )(xt, nt)


def _sc_gather(tbl, idxg):
    total = idxg.shape[0]
    d = tbl.shape[1]
    per_w = total // _NW
    n_ch = per_w // _CH
    mesh = plsc.VectorSubcoreMesh(core_axis_name="c", subcore_axis_name="s")

    @functools.partial(
        pl.kernel,
        mesh=mesh,
        out_type=jax.ShapeDtypeStruct((total, d), jnp.float32),
        scratch_types=[
            pltpu.VMEM((_CH,), jnp.int32),
            pltpu.VMEM((_CH, d), jnp.float32),
            pltpu.SemaphoreType.DMA,
        ],
    )
    def gk(tbl_hbm, idx_hbm, out_hbm, idx_v, rows_v, sem):
        wid = lax.axis_index("s") * 2 + lax.axis_index("c")
        base = wid * per_w

        def step(j, carry):
            off = base + j * _CH
            pltpu.sync_copy(idx_hbm.at[pl.ds(off, _CH)], idx_v)
            pltpu.async_copy(tbl_hbm.at[idx_v], rows_v, sem).wait()
            pltpu.sync_copy(rows_v, out_hbm.at[pl.ds(off, _CH)])
            return carry

        lax.fori_loop(0, n_ch, step, 0)

    return gk(tbl, idxg)


def _mlp_body(g_ref, nx_ref, w1_ref, w1c_ref, b1_ref, w2_ref, b2_ref,
              w3_ref, b3_ref, out_ref):
    K = _NSAMPLE
    g = g_ref[0]
    h = jnp.dot(g, w1_ref[...], preferred_element_type=jnp.float32)
    c = nx_ref[0]
    ct = jnp.dot(c, w1c_ref[...], preferred_element_type=jnp.float32)
    h = h.reshape(_TS, K, h.shape[-1]) - ct[:, None, :]
    h = jnp.maximum(h + b1_ref[...][None], 0.0)
    h = h.reshape(_TS * K, h.shape[-1])
    h = jnp.maximum(
        jnp.dot(h, w2_ref[...], preferred_element_type=jnp.float32)
        + b2_ref[...], 0.0)
    h = jnp.maximum(
        jnp.dot(h, w3_ref[...], preferred_element_type=jnp.float32)
        + b3_ref[...], 0.0)
    out_ref[0] = jnp.max(h.reshape(_TS, K, h.shape[-1]), axis=1)


def _mlp(g, nxp, wbig, w1c, b1, w2, b2, w3, b3):
    B = g.shape[0]
    S = nxp.shape[1]
    D = g.shape[2]
    C3 = w3.shape[1]
    full = lambda shp: pl.BlockSpec(shp, lambda b, t: tuple(0 for _ in shp))
    return pl.pallas_call(
        _mlp_body,
        grid=(B, S // _TS),
        in_specs=[
            pl.BlockSpec((1, _TS * _NSAMPLE, D), lambda b, t: (b, t, 0)),
            pl.BlockSpec((1, _TS, 8), lambda b, t: (b, t, 0)),
            full(wbig.shape),
            full(w1c.shape),
            full(b1.shape),
            full(w2.shape),
            full(b2.shape),
            full(w3.shape),
            full(b3.shape),
        ],
        out_specs=pl.BlockSpec((1, _TS, C3), lambda b, t: (b, t, 0)),
        out_shape=jax.ShapeDtypeStruct((B, S, C3), jnp.float32),
    )(g, nxp, wbig, w1c, b1, w2, b2, w3, b3)


def kernel(xyz, features, W1, b1, W2, b2, W3, b3):
    B, N, _ = xyz.shape
    C = features.shape[1]
    S, K = _NPOINT, _NSAMPLE
    f32 = jnp.float32

    xt = jnp.transpose(xyz, (0, 2, 1))
    nx_pad = _fps(xt)
    new_xyz = nx_pad[:, :, :3]
    nt = jnp.transpose(new_xyz, (0, 2, 1))
    idx = _ball_query(xt, nt)

    pad = (-(C + 3)) % 128
    D = C + 3 + pad
    feats_t = jnp.transpose(features, (0, 2, 1))
    tbl = jnp.concatenate(
        [feats_t, xyz, jnp.zeros((B, N, pad), f32)], axis=-1
    ).reshape(B * N, D)
    idxg = (idx + (jnp.arange(B, dtype=jnp.int32) * N)[:, None, None]
            ).reshape(-1)
    g = _sc_gather(tbl, idxg).reshape(B, S * K, D)

    nxp = jnp.concatenate([new_xyz, jnp.zeros((B, S, 5), f32)], axis=-1)
    wbig = jnp.concatenate(
        [W1[3:], W1[:3], jnp.zeros((pad, W1.shape[1]), f32)], axis=0)
    w1c = jnp.concatenate([W1[:3], jnp.zeros((5, W1.shape[1]), f32)], axis=0)
    out = _mlp(g, nxp, wbig, w1c, b1.reshape(1, -1), W2, b2.reshape(1, -1),
               W3, b3.reshape(1, -1))
    new_features = jnp.transpose(out, (0, 2, 1))
    return (new_xyz, new_features)


# speedup vs baseline: 1.5165x; 1.0016x over previous
import functools

import jax
import jax.numpy as jnp
import numpy as np
from jax import lax
from jax.experimental import pallas as pl
from jax.experimental.pallas import tpu as pltpu
from jax.experimental.pallas import tpu_sc as plsc

_NPOINT = 512
_NSAMPLE = 32
_RADIUS2 = np.float32(0.2 ** 2)
_TS = 128
_NW = 32
_CH = 128


def _fps_body(xt_ref, out_ref):
    B = xt_ref.shape[0]
    N = xt_ref.shape[2]
    R = N // 128
    xs, ys, zs, xall = [], [], [], []
    for b in range(B):
        x = xt_ref[b, 0, :].reshape(R, 128)
        y = xt_ref[b, 1, :].reshape(R, 128)
        z = xt_ref[b, 2, :].reshape(R, 128)
        xs.append(x); ys.append(y); zs.append(z)
        xall.append(jnp.concatenate([x, y, z], axis=1))
    iota = (lax.broadcasted_iota(jnp.int32, (R, 128), 0) * 128
            + lax.broadcasted_iota(jnp.int32, (R, 128), 1))
    lane = lax.broadcasted_iota(jnp.int32, (1, 128), 1)
    rowc = lax.broadcasted_iota(jnp.int32, (R, 1), 0)
    oh0 = (lane == 0).astype(jnp.float32)
    oh1 = (lane == 1).astype(jnp.float32)
    oh2 = (lane == 2).astype(jnp.float32)

    def gather3(nxt, b):
        r = lax.shift_right_logical(nxt, 7)
        l = jnp.bitwise_and(nxt, 127)
        oneR = (rowc == r).astype(jnp.float32)
        oneL = (lane == l).astype(jnp.float32)
        tmp = jnp.sum(xall[b] * oneR, axis=0, keepdims=True)
        cx = jnp.sum(tmp[:, :128] * oneL, axis=1, keepdims=True)
        cy = jnp.sum(tmp[:, 128:256] * oneL, axis=1, keepdims=True)
        cz = jnp.sum(tmp[:, 256:] * oneL, axis=1, keepdims=True)
        return cx, cy, cz

    dist0, c0 = [], []
    for b in range(B):
        dist0.append(jnp.full((R, 128), 1e10, jnp.float32))
        c0.append((xs[b][0:1, 0:1], ys[b][0:1, 0:1], zs[b][0:1, 0:1]))

    def body(i, carry):
        new = []
        for b in range(B):
            dist, (cx, cy, cz) = carry[b]
            row = cx * oh0 + cy * oh1 + cz * oh2
            out_ref[b, pl.ds(i - 1, 1), :] = row
            dx = xs[b] - cx
            dy = ys[b] - cy
            dz = zs[b] - cz
            d = dx * dx + dy * dy
            d = d + dz * dz
            dist = jnp.minimum(dist, d)
            m = jnp.max(dist, axis=0, keepdims=True)
            m = jnp.max(m, axis=1, keepdims=True)
            cand = jnp.where(dist == m, iota, N)
            nxt = jnp.min(cand, axis=0, keepdims=True)
            nxt = jnp.min(nxt, axis=1, keepdims=True)
            new.append((dist, gather3(nxt, b)))
        return tuple(new)

    carry = lax.fori_loop(1, _NPOINT, body,
                          tuple((dist0[b], c0[b]) for b in range(B)))
    for b in range(B):
        _, (cx, cy, cz) = carry[b]
        row = cx * oh0 + cy * oh1 + cz * oh2
        out_ref[b, pl.ds(_NPOINT - 1, 1), :] = row


def _fps(xt):
    B, _, N = xt.shape
    return pl.pallas_call(
        _fps_body,
        out_shape=jax.ShapeDtypeStruct((B, _NPOINT, 128), jnp.float32),
    )(xt)


def _bq_body(xt_ref, nt_ref, out_ref):
    N = xt_ref.shape[2]
    x = xt_ref[0, 0, :]
    y = xt_ref[0, 1, :]
    z = xt_ref[0, 2, :]
    cx = nt_ref[0, 0, :]
    cy = nt_ref[0, 1, :]
    cz = nt_ref[0, 2, :]
    dx = cx[:, None] - x[None, :]
    dy = cy[:, None] - y[None, :]
    dz = cz[:, None] - z[None, :]
    d2 = dx * dx + dy * dy
    d2 = d2 + dz * dz
    iota = lax.broadcasted_iota(jnp.int32, (_TS, N), 1)
    cand = jnp.where(d2 < _RADIUS2, iota, N)
    first = jnp.min(cand, axis=1)
    cols = []
    cur = cand
    for _ in range(_NSAMPLE):
        v = jnp.min(cur, axis=1)
        cols.append(jnp.where(v == N, first, v)[:, None])
        cur = jnp.where(cur == v[:, None], N, cur)
    out_ref[0] = jnp.concatenate(cols, axis=1)


def _ball_query(xt, nt):
    B, _, N = xt.shape
    S = nt.shape[2]
    return pl.pallas_call(
        _bq_body,
        grid=(B, S // _TS),
        in_specs=[
            pl.BlockSpec((1, 3, N), lambda b, t: (b, 0, 0)),
            pl.BlockSpec((1, 3, _TS), lambda b, t: (b, 0, t)),
        ],
        out_specs=pl.BlockSpec((1, _TS, _NSAMPLE), lambda b, t: (b, t, 0)),
        out_shape=jax.ShapeDtypeStruct((B, S, _NSAMPLE), jnp.int32),
    )(xt, nt)


def _sc_gather(tbl, idxg):
    total = idxg.shape[0]
    d = tbl.shape[1]
    per_w = total // _NW
    n_ch = per_w // _CH
    mesh = plsc.VectorSubcoreMesh(core_axis_name="c", subcore_axis_name="s")

    @functools.partial(
        pl.kernel,
        mesh=mesh,
        out_type=jax.ShapeDtypeStruct((total, d), jnp.float32),
        scratch_types=[
            pltpu.VMEM((_CH,), jnp.int32),
            pltpu.VMEM((_CH, d), jnp.float32),
            pltpu.SemaphoreType.DMA,
        ],
    )
    def gk(tbl_hbm, idx_hbm, out_hbm, idx_v, rows_v, sem):
        wid = lax.axis_index("s") * 2 + lax.axis_index("c")
        base = wid * per_w

        def step(j, carry):
            off = base + j * _CH
            pltpu.sync_copy(idx_hbm.at[pl.ds(off, _CH)], idx_v)
            pltpu.async_copy(tbl_hbm.at[idx_v], rows_v, sem).wait()
            pltpu.sync_copy(rows_v, out_hbm.at[pl.ds(off, _CH)])
            return carry

        lax.fori_loop(0, n_ch, step, 0)

    return gk(tbl, idxg)


def _mlp_body(g_ref, nx_ref, w1_ref, w1c_ref, b1_ref, w2_ref, b2_ref,
              w3_ref, b3_ref, out_ref):
    K = _NSAMPLE
    g = g_ref[0]
    h = jnp.dot(g, w1_ref[...], preferred_element_type=jnp.float32)
    c = nx_ref[0]
    ct = jnp.dot(c, w1c_ref[...], preferred_element_type=jnp.float32)
    h = h.reshape(_TS, K, h.shape[-1]) - ct[:, None, :]
    h = jnp.maximum(h + b1_ref[...][None], 0.0)
    h = h.reshape(_TS * K, h.shape[-1])
    h = jnp.maximum(
        jnp.dot(h, w2_ref[...], preferred_element_type=jnp.float32)
        + b2_ref[...], 0.0)
    h = jnp.maximum(
        jnp.dot(h, w3_ref[...], preferred_element_type=jnp.float32)
        + b3_ref[...], 0.0)
    out_ref[0] = jnp.max(h.reshape(_TS, K, h.shape[-1]), axis=1)


def _mlp(g, nxp, wbig, w1c, b1, w2, b2, w3, b3):
    B = g.shape[0]
    S = nxp.shape[1]
    D = g.shape[2]
    C3 = w3.shape[1]
    full = lambda shp: pl.BlockSpec(shp, lambda b, t: tuple(0 for _ in shp))
    return pl.pallas_call(
        _mlp_body,
        grid=(B, S // _TS),
        in_specs=[
            pl.BlockSpec((1, _TS * _NSAMPLE, D), lambda b, t: (b, t, 0)),
            pl.BlockSpec((1, _TS, 8), lambda b, t: (b, t, 0)),
            full(wbig.shape),
            full(w1c.shape),
            full(b1.shape),
            full(w2.shape),
            full(b2.shape),
            full(w3.shape),
            full(b3.shape),
        ],
        out_specs=pl.BlockSpec((1, _TS, C3), lambda b, t: (b, t, 0)),
        out_shape=jax.ShapeDtypeStruct((B, S, C3), jnp.float32),
    )(g, nxp, wbig, w1c, b1, w2, b2, w3, b3)


def kernel(xyz, features, W1, b1, W2, b2, W3, b3):
    B, N, _ = xyz.shape
    C = features.shape[1]
    S, K = _NPOINT, _NSAMPLE
    f32 = jnp.float32

    xt = jnp.transpose(xyz, (0, 2, 1))
    nx_pad = _fps(xt)
    new_xyz = nx_pad[:, :, :3]
    nt = jnp.transpose(new_xyz, (0, 2, 1))
    idx = _ball_query(xt, nt)

    pad = (-(C + 3)) % 128
    D = C + 3 + pad
    feats_t = jnp.transpose(features, (0, 2, 1))
    tbl = jnp.concatenate(
        [feats_t, xyz, jnp.zeros((B, N, pad), f32)], axis=-1
    ).reshape(B * N, D)
    idxg = (idx + (jnp.arange(B, dtype=jnp.int32) * N)[:, None, None]
            ).reshape(-1)
    g = _sc_gather(tbl, idxg).reshape(B, S * K, D)

    nxp = jnp.concatenate([new_xyz, jnp.zeros((B, S, 5), f32)], axis=-1)
    wbig = jnp.concatenate(
        [W1[3:], W1[:3], jnp.zeros((pad, W1.shape[1]), f32)], axis=0)
    w1c = jnp.concatenate([W1[:3], jnp.zeros((5, W1.shape[1]), f32)], axis=0)
    out = _mlp(g, nxp, wbig, w1c, b1.reshape(1, -1), W2, b2.reshape(1, -1),
               W3, b3.reshape(1, -1))
    new_features = jnp.transpose(out, (0, 2, 1))
    return (new_xyz, new_features)


# speedup vs baseline: 1.6894x; 1.1140x over previous
import functools

import jax
import jax.numpy as jnp
import numpy as np
from jax import lax
from jax.experimental import pallas as pl
from jax.experimental.pallas import tpu as pltpu
from jax.experimental.pallas import tpu_sc as plsc

_NPOINT = 512
_NSAMPLE = 32
_RADIUS2 = np.float32(0.2 ** 2)
_TS = 128
_NW = 32
_CH = 128


def _fps_body(xt_ref, out_ref):
    B = xt_ref.shape[0]
    N = xt_ref.shape[2]
    R = N // 128
    xs, ys, zs, xall = [], [], [], []
    for b in range(B):
        x = xt_ref[b, 0, :].reshape(R, 128)
        y = xt_ref[b, 1, :].reshape(R, 128)
        z = xt_ref[b, 2, :].reshape(R, 128)
        xs.append(x); ys.append(y); zs.append(z)
        xall.append(jnp.concatenate([x, y, z], axis=1))
    iota = (lax.broadcasted_iota(jnp.int32, (R, 128), 0) * 128
            + lax.broadcasted_iota(jnp.int32, (R, 128), 1))
    lane = lax.broadcasted_iota(jnp.int32, (1, 128), 1)
    rowc = lax.broadcasted_iota(jnp.int32, (R, 1), 0)
    oh0 = (lane == 0).astype(jnp.float32)
    oh1 = (lane == 1).astype(jnp.float32)
    oh2 = (lane == 2).astype(jnp.float32)

    def gather3(nxt, b):
        r = lax.shift_right_logical(nxt, 7)
        l = jnp.bitwise_and(nxt, 127)
        oneR = (rowc == r).astype(jnp.float32)
        oneL = (lane == l).astype(jnp.float32)
        tmp = jnp.sum(xall[b] * oneR, axis=0, keepdims=True)
        cx = jnp.sum(tmp[:, :128] * oneL, axis=1, keepdims=True)
        cy = jnp.sum(tmp[:, 128:256] * oneL, axis=1, keepdims=True)
        cz = jnp.sum(tmp[:, 256:] * oneL, axis=1, keepdims=True)
        return cx, cy, cz

    dist0, c0 = [], []
    for b in range(B):
        dist0.append(jnp.full((R, 128), 1e10, jnp.float32))
        c0.append((xs[b][0:1, 0:1], ys[b][0:1, 0:1], zs[b][0:1, 0:1]))

    def body(i, carry):
        new = []
        for b in range(B):
            dist, (cx, cy, cz) = carry[b]
            row = cx * oh0 + cy * oh1 + cz * oh2
            out_ref[b, pl.ds(i - 1, 1), :] = row
            dx = xs[b] - cx
            dy = ys[b] - cy
            dz = zs[b] - cz
            d = dx * dx + dy * dy
            d = d + dz * dz
            dist = jnp.minimum(dist, d)
            m = jnp.max(dist, axis=0, keepdims=True)
            m = jnp.max(m, axis=1, keepdims=True)
            cand = jnp.where(dist == m, iota, N)
            nxt = jnp.min(cand, axis=0, keepdims=True)
            nxt = jnp.min(nxt, axis=1, keepdims=True)
            new.append((dist, gather3(nxt, b)))
        return tuple(new)

    carry = lax.fori_loop(1, _NPOINT, body,
                          tuple((dist0[b], c0[b]) for b in range(B)))
    for b in range(B):
        _, (cx, cy, cz) = carry[b]
        row = cx * oh0 + cy * oh1 + cz * oh2
        out_ref[b, pl.ds(_NPOINT - 1, 1), :] = row


def _fps(xt):
    B, _, N = xt.shape
    return pl.pallas_call(
        _fps_body,
        out_shape=jax.ShapeDtypeStruct((B, _NPOINT, 128), jnp.float32),
    )(xt)


def _bq_body(xt_ref, nt_ref, out_ref):
    N = xt_ref.shape[2]
    x = xt_ref[0, 0, :]
    y = xt_ref[0, 1, :]
    z = xt_ref[0, 2, :]
    cx = nt_ref[0, 0, :]
    cy = nt_ref[0, 1, :]
    cz = nt_ref[0, 2, :]
    dx = cx[:, None] - x[None, :]
    dy = cy[:, None] - y[None, :]
    dz = cz[:, None] - z[None, :]
    d2 = dx * dx + dy * dy
    d2 = d2 + dz * dz
    iota = lax.broadcasted_iota(jnp.int32, (_TS, N), 1)
    cand = jnp.where(d2 < _RADIUS2, iota, N)
    Q = N // 4
    a0 = cand[:, :Q]
    a1 = cand[:, Q:2 * Q]
    a2 = cand[:, 2 * Q:3 * Q]
    a3 = cand[:, 3 * Q:]
    lo1, hi1 = jnp.minimum(a0, a1), jnp.maximum(a0, a1)
    lo2, hi2 = jnp.minimum(a2, a3), jnp.maximum(a2, a3)
    s0, t0 = jnp.minimum(lo1, lo2), jnp.maximum(lo1, lo2)
    s3, t1 = jnp.maximum(hi1, hi2), jnp.minimum(hi1, hi2)
    s1, s2 = jnp.minimum(t0, t1), jnp.maximum(t0, t1)
    cols = []
    first = None
    for k in range(_NSAMPLE):
        v = jnp.min(s0, axis=1)
        if k == 0:
            first = v
            cols.append(v[:, None])
        else:
            cols.append(jnp.where(v == N, first, v)[:, None])
        sel = s0 == v[:, None]
        s0 = jnp.where(sel, s1, s0)
        s1 = jnp.where(sel, s2, s1)
        s2 = jnp.where(sel, s3, s2)
        s3 = jnp.where(sel, N, s3)
    out_ref[0] = jnp.concatenate(cols, axis=1)


def _ball_query(xt, nt):
    B, _, N = xt.shape
    S = nt.shape[2]
    return pl.pallas_call(
        _bq_body,
        grid=(B, S // _TS),
        in_specs=[
            pl.BlockSpec((1, 3, N), lambda b, t: (b, 0, 0)),
            pl.BlockSpec((1, 3, _TS), lambda b, t: (b, 0, t)),
        ],
        out_specs=pl.BlockSpec((1, _TS, _NSAMPLE), lambda b, t: (b, t, 0)),
        out_shape=jax.ShapeDtypeStruct((B, S, _NSAMPLE), jnp.int32),
    )(xt, nt)


def _sc_gather(tbl, idxg):
    total = idxg.shape[0]
    d = tbl.shape[1]
    per_w = total // _NW
    n_ch = per_w // _CH
    mesh = plsc.VectorSubcoreMesh(core_axis_name="c", subcore_axis_name="s")

    @functools.partial(
        pl.kernel,
        mesh=mesh,
        out_type=jax.ShapeDtypeStruct((total, d), jnp.float32),
        scratch_types=[
            pltpu.VMEM((_CH,), jnp.int32),
            pltpu.VMEM((_CH, d), jnp.float32),
            pltpu.SemaphoreType.DMA,
        ],
    )
    def gk(tbl_hbm, idx_hbm, out_hbm, idx_v, rows_v, sem):
        wid = lax.axis_index("s") * 2 + lax.axis_index("c")
        base = wid * per_w

        def step(j, carry):
            off = base + j * _CH
            pltpu.sync_copy(idx_hbm.at[pl.ds(off, _CH)], idx_v)
            pltpu.async_copy(tbl_hbm.at[idx_v], rows_v, sem).wait()
            pltpu.sync_copy(rows_v, out_hbm.at[pl.ds(off, _CH)])
            return carry

        lax.fori_loop(0, n_ch, step, 0)

    return gk(tbl, idxg)


def _mlp_body(g_ref, nx_ref, w1_ref, w1c_ref, b1_ref, w2_ref, b2_ref,
              w3_ref, b3_ref, out_ref):
    K = _NSAMPLE
    g = g_ref[0]
    h = jnp.dot(g, w1_ref[...], preferred_element_type=jnp.float32)
    c = nx_ref[0]
    ct = jnp.dot(c, w1c_ref[...], preferred_element_type=jnp.float32)
    h = h.reshape(_TS, K, h.shape[-1]) - ct[:, None, :]
    h = jnp.maximum(h + b1_ref[...][None], 0.0)
    h = h.reshape(_TS * K, h.shape[-1])
    h = jnp.maximum(
        jnp.dot(h, w2_ref[...], preferred_element_type=jnp.float32)
        + b2_ref[...], 0.0)
    h = jnp.maximum(
        jnp.dot(h, w3_ref[...], preferred_element_type=jnp.float32)
        + b3_ref[...], 0.0)
    out_ref[0] = jnp.max(h.reshape(_TS, K, h.shape[-1]), axis=1)


def _mlp(g, nxp, wbig, w1c, b1, w2, b2, w3, b3):
    B = g.shape[0]
    S = nxp.shape[1]
    D = g.shape[2]
    C3 = w3.shape[1]
    full = lambda shp: pl.BlockSpec(shp, lambda b, t: tuple(0 for _ in shp))
    return pl.pallas_call(
        _mlp_body,
        grid=(B, S // _TS),
        in_specs=[
            pl.BlockSpec((1, _TS * _NSAMPLE, D), lambda b, t: (b, t, 0)),
            pl.BlockSpec((1, _TS, 8), lambda b, t: (b, t, 0)),
            full(wbig.shape),
            full(w1c.shape),
            full(b1.shape),
            full(w2.shape),
            full(b2.shape),
            full(w3.shape),
            full(b3.shape),
        ],
        out_specs=pl.BlockSpec((1, _TS, C3), lambda b, t: (b, t, 0)),
        out_shape=jax.ShapeDtypeStruct((B, S, C3), jnp.float32),
    )(g, nxp, wbig, w1c, b1, w2, b2, w3, b3)


def kernel(xyz, features, W1, b1, W2, b2, W3, b3):
    B, N, _ = xyz.shape
    C = features.shape[1]
    S, K = _NPOINT, _NSAMPLE
    f32 = jnp.float32

    xt = jnp.transpose(xyz, (0, 2, 1))
    nx_pad = _fps(xt)
    new_xyz = nx_pad[:, :, :3]
    nt = jnp.transpose(new_xyz, (0, 2, 1))
    idx = _ball_query(xt, nt)

    pad = (-(C + 3)) % 128
    D = C + 3 + pad
    feats_t = jnp.transpose(features, (0, 2, 1))
    tbl = jnp.concatenate(
        [feats_t, xyz, jnp.zeros((B, N, pad), f32)], axis=-1
    ).reshape(B * N, D)
    idxg = (idx + (jnp.arange(B, dtype=jnp.int32) * N)[:, None, None]
            ).reshape(-1)
    g = _sc_gather(tbl, idxg).reshape(B, S * K, D)

    nxp = jnp.concatenate([new_xyz, jnp.zeros((B, S, 5), f32)], axis=-1)
    wbig = jnp.concatenate(
        [W1[3:], W1[:3], jnp.zeros((pad, W1.shape[1]), f32)], axis=0)
    w1c = jnp.concatenate([W1[:3], jnp.zeros((5, W1.shape[1]), f32)], axis=0)
    out = _mlp(g, nxp, wbig, w1c, b1.reshape(1, -1), W2, b2.reshape(1, -1),
               W3, b3.reshape(1, -1))
    new_features = jnp.transpose(out, (0, 2, 1))
    return (new_xyz, new_features)


# speedup vs baseline: 2.2648x; 1.3406x over previous
import functools

import jax
import jax.numpy as jnp
import numpy as np
from jax import lax
from jax.experimental import pallas as pl
from jax.experimental.pallas import tpu as pltpu
from jax.experimental.pallas import tpu_sc as plsc

_NPOINT = 512
_NSAMPLE = 32
_RADIUS2 = np.float32(0.2 ** 2)
_TS = 128
_NW = 32
_CH = 128


def _fps_body(xt_ref, out_ref):
    B = xt_ref.shape[0]
    N = xt_ref.shape[2]
    R = N // 128
    xs, ys, zs, xall = [], [], [], []
    for b in range(B):
        x = xt_ref[b, 0, :].reshape(R, 128)
        y = xt_ref[b, 1, :].reshape(R, 128)
        z = xt_ref[b, 2, :].reshape(R, 128)
        xs.append(x); ys.append(y); zs.append(z)
        xall.append(jnp.concatenate([x, y, z], axis=1))
    iota = (lax.broadcasted_iota(jnp.int32, (R, 128), 0) * 128
            + lax.broadcasted_iota(jnp.int32, (R, 128), 1))
    lane = lax.broadcasted_iota(jnp.int32, (1, 128), 1)
    rowc = lax.broadcasted_iota(jnp.int32, (R, 1), 0)
    oh0 = (lane == 0).astype(jnp.float32)
    oh1 = (lane == 1).astype(jnp.float32)
    oh2 = (lane == 2).astype(jnp.float32)

    def gather3(nxt, b):
        r = lax.shift_right_logical(nxt, 7)
        l = jnp.bitwise_and(nxt, 127)
        oneR = (rowc == r).astype(jnp.float32)
        oneL = (lane == l).astype(jnp.float32)
        tmp = jnp.sum(xall[b] * oneR, axis=0, keepdims=True)
        cx = jnp.sum(tmp[:, :128] * oneL, axis=1, keepdims=True)
        cy = jnp.sum(tmp[:, 128:256] * oneL, axis=1, keepdims=True)
        cz = jnp.sum(tmp[:, 256:] * oneL, axis=1, keepdims=True)
        return cx, cy, cz

    dist0, c0 = [], []
    for b in range(B):
        dist0.append(jnp.full((R, 128), 1e10, jnp.float32))
        c0.append((xs[b][0:1, 0:1], ys[b][0:1, 0:1], zs[b][0:1, 0:1]))

    def body(i, carry):
        new = []
        for b in range(B):
            dist, (cx, cy, cz) = carry[b]
            row = cx * oh0 + cy * oh1 + cz * oh2
            out_ref[b, pl.ds(i - 1, 1), :] = row
            dx = xs[b] - cx
            dy = ys[b] - cy
            dz = zs[b] - cz
            d = dx * dx + dy * dy
            d = d + dz * dz
            dist = jnp.minimum(dist, d)
            m = jnp.max(dist, axis=0, keepdims=True)
            m = jnp.max(m, axis=1, keepdims=True)
            cand = jnp.where(dist == m, iota, N)
            nxt = jnp.min(cand, axis=0, keepdims=True)
            nxt = jnp.min(nxt, axis=1, keepdims=True)
            new.append((dist, gather3(nxt, b)))
        return tuple(new)

    carry = lax.fori_loop(1, _NPOINT, body,
                          tuple((dist0[b], c0[b]) for b in range(B)))
    for b in range(B):
        _, (cx, cy, cz) = carry[b]
        row = cx * oh0 + cy * oh1 + cz * oh2
        out_ref[b, pl.ds(_NPOINT - 1, 1), :] = row


def _fps(xt):
    B, _, N = xt.shape
    return pl.pallas_call(
        _fps_body,
        out_shape=jax.ShapeDtypeStruct((B, _NPOINT, 128), jnp.float32),
    )(xt)


def _bq_body(xyz_ref, nt_ref, out_ref):
    N = xyz_ref.shape[1]
    W = N // 32
    x = xyz_ref[0, :, 0:1]
    y = xyz_ref[0, :, 1:2]
    z = xyz_ref[0, :, 2:3]
    cx = nt_ref[0, 0, :][None, :]
    cy = nt_ref[0, 1, :][None, :]
    cz = nt_ref[0, 2, :][None, :]
    dx = cx - x
    dy = cy - y
    dz = cz - z
    d2 = dx * dx + dy * dy
    d2 = d2 + dz * dz
    sub = lax.broadcasted_iota(jnp.int32, (N, 1), 0)
    pw = jnp.left_shift(jnp.int32(1), jnp.bitwise_and(sub, 31))
    bits = jnp.where(d2 < _RADIUS2, pw, 0)
    words = jnp.sum(bits.reshape(W, 32, _TS), axis=1)
    rowbase = lax.broadcasted_iota(jnp.int32, (W, _TS), 0) * 32
    rowi = lax.broadcasted_iota(jnp.int32, (W, _TS), 0)
    cols = []
    first = None
    for k in range(_NSAMPLE):
        b = jnp.bitwise_and(words, -words)
        e = lax.shift_right_logical(
            lax.bitcast_convert_type(b.astype(jnp.float32), jnp.int32), 23)
        bit = jnp.bitwise_and(e, 255) - 127
        cand = jnp.where(words != 0, rowbase + bit, N)
        v = jnp.min(cand, axis=0, keepdims=True)
        if k == 0:
            first = v
            cols.append(v)
        else:
            cols.append(jnp.where(v == N, first, v))
        wsel = rowi == lax.shift_right_logical(v, 5)
        pat = jnp.left_shift(jnp.int32(1), jnp.bitwise_and(v, 31))
        words = jnp.where(wsel, jnp.bitwise_xor(words, pat), words)
    out_ref[0] = jnp.concatenate(cols, axis=0)


def _ball_query(xyz, nt):
    B, N, _ = xyz.shape
    S = nt.shape[2]
    return pl.pallas_call(
        _bq_body,
        grid=(B, S // _TS),
        in_specs=[
            pl.BlockSpec((1, N, 3), lambda b, t: (b, 0, 0)),
            pl.BlockSpec((1, 3, _TS), lambda b, t: (b, 0, t)),
        ],
        out_specs=pl.BlockSpec((1, _NSAMPLE, _TS), lambda b, t: (b, 0, t)),
        out_shape=jax.ShapeDtypeStruct((B, _NSAMPLE, S), jnp.int32),
    )(xyz, nt)


def _sc_gather(tbl, idxg):
    total = idxg.shape[0]
    d = tbl.shape[1]
    per_w = total // _NW
    n_ch = per_w // _CH
    mesh = plsc.VectorSubcoreMesh(core_axis_name="c", subcore_axis_name="s")

    @functools.partial(
        pl.kernel,
        mesh=mesh,
        out_type=jax.ShapeDtypeStruct((total, d), jnp.float32),
        scratch_types=[
            pltpu.VMEM((_CH,), jnp.int32),
            pltpu.VMEM((_CH, d), jnp.float32),
            pltpu.SemaphoreType.DMA,
        ],
    )
    def gk(tbl_hbm, idx_hbm, out_hbm, idx_v, rows_v, sem):
        wid = lax.axis_index("s") * 2 + lax.axis_index("c")
        base = wid * per_w

        def step(j, carry):
            off = base + j * _CH
            pltpu.sync_copy(idx_hbm.at[pl.ds(off, _CH)], idx_v)
            pltpu.async_copy(tbl_hbm.at[idx_v], rows_v, sem).wait()
            pltpu.sync_copy(rows_v, out_hbm.at[pl.ds(off, _CH)])
            return carry

        lax.fori_loop(0, n_ch, step, 0)

    return gk(tbl, idxg)


def _mlp_body(g_ref, nx_ref, w1_ref, w1c_ref, b1_ref, w2_ref, b2_ref,
              w3_ref, b3_ref, out_ref):
    K = _NSAMPLE
    g = g_ref[0]
    h = jnp.dot(g, w1_ref[...], preferred_element_type=jnp.float32)
    c = nx_ref[0]
    ct = jnp.dot(c, w1c_ref[...], preferred_element_type=jnp.float32)
    h = h.reshape(_TS, K, h.shape[-1]) - ct[:, None, :]
    h = jnp.maximum(h + b1_ref[...][None], 0.0)
    h = h.reshape(_TS * K, h.shape[-1])
    h = jnp.maximum(
        jnp.dot(h, w2_ref[...], preferred_element_type=jnp.float32)
        + b2_ref[...], 0.0)
    h = jnp.maximum(
        jnp.dot(h, w3_ref[...], preferred_element_type=jnp.float32)
        + b3_ref[...], 0.0)
    out_ref[0] = jnp.max(h.reshape(_TS, K, h.shape[-1]), axis=1)


def _mlp(g, nxp, wbig, w1c, b1, w2, b2, w3, b3):
    B = g.shape[0]
    S = nxp.shape[1]
    D = g.shape[2]
    C3 = w3.shape[1]
    full = lambda shp: pl.BlockSpec(shp, lambda b, t: tuple(0 for _ in shp))
    return pl.pallas_call(
        _mlp_body,
        grid=(B, S // _TS),
        in_specs=[
            pl.BlockSpec((1, _TS * _NSAMPLE, D), lambda b, t: (b, t, 0)),
            pl.BlockSpec((1, _TS, 8), lambda b, t: (b, t, 0)),
            full(wbig.shape),
            full(w1c.shape),
            full(b1.shape),
            full(w2.shape),
            full(b2.shape),
            full(w3.shape),
            full(b3.shape),
        ],
        out_specs=pl.BlockSpec((1, _TS, C3), lambda b, t: (b, t, 0)),
        out_shape=jax.ShapeDtypeStruct((B, S, C3), jnp.float32),
    )(g, nxp, wbig, w1c, b1, w2, b2, w3, b3)


def kernel(xyz, features, W1, b1, W2, b2, W3, b3):
    B, N, _ = xyz.shape
    C = features.shape[1]
    S, K = _NPOINT, _NSAMPLE
    f32 = jnp.float32

    xt = jnp.transpose(xyz, (0, 2, 1))
    nx_pad = _fps(xt)
    new_xyz = nx_pad[:, :, :3]
    nt = jnp.transpose(new_xyz, (0, 2, 1))
    idx = jnp.transpose(_ball_query(xyz, nt), (0, 2, 1))

    pad = (-(C + 3)) % 128
    D = C + 3 + pad
    feats_t = jnp.transpose(features, (0, 2, 1))
    tbl = jnp.concatenate(
        [feats_t, xyz, jnp.zeros((B, N, pad), f32)], axis=-1
    ).reshape(B * N, D)
    idxg = (idx + (jnp.arange(B, dtype=jnp.int32) * N)[:, None, None]
            ).reshape(-1)
    g = _sc_gather(tbl, idxg).reshape(B, S * K, D)

    nxp = jnp.concatenate([new_xyz, jnp.zeros((B, S, 5), f32)], axis=-1)
    wbig = jnp.concatenate(
        [W1[3:], W1[:3], jnp.zeros((pad, W1.shape[1]), f32)], axis=0)
    w1c = jnp.concatenate([W1[:3], jnp.zeros((5, W1.shape[1]), f32)], axis=0)
    out = _mlp(g, nxp, wbig, w1c, b1.reshape(1, -1), W2, b2.reshape(1, -1),
               W3, b3.reshape(1, -1))
    new_features = jnp.transpose(out, (0, 2, 1))
    return (new_xyz, new_features)


# speedup vs baseline: 2.4019x; 1.0605x over previous
import functools

import jax
import jax.numpy as jnp
import numpy as np
from jax import lax
from jax.experimental import pallas as pl
from jax.experimental.pallas import tpu as pltpu
from jax.experimental.pallas import tpu_sc as plsc

_NPOINT = 512
_NSAMPLE = 32
_RADIUS2 = np.float32(0.2 ** 2)
_TS = 128
_NW = 32
_CH = 128


def _fps_body(xt_ref, out_ref):
    B = xt_ref.shape[0]
    N = xt_ref.shape[2]
    R = N // 128
    xs, ys, zs, xall = [], [], [], []
    for b in range(B):
        x = xt_ref[b, 0, :].reshape(R, 128)
        y = xt_ref[b, 1, :].reshape(R, 128)
        z = xt_ref[b, 2, :].reshape(R, 128)
        xs.append(x); ys.append(y); zs.append(z)
        xall.append(jnp.concatenate([x, y, z], axis=1))
    rowiota = lax.broadcasted_iota(jnp.int32, (R, 128), 0)
    lane = lax.broadcasted_iota(jnp.int32, (1, 128), 1)
    rowc = lax.broadcasted_iota(jnp.int32, (R, 1), 0)
    oh0 = (lane == 0).astype(jnp.float32)
    oh1 = (lane == 1).astype(jnp.float32)
    oh2 = (lane == 2).astype(jnp.float32)

    def gather3(nxt, b):
        r = lax.shift_right_logical(nxt, 7)
        l = jnp.bitwise_and(nxt, 127)
        oneR = (rowc == r).astype(jnp.float32)
        oneL = (lane == l).astype(jnp.float32)
        tmp = jnp.sum(xall[b] * oneR, axis=0, keepdims=True)
        cx = jnp.sum(tmp[:, :128] * oneL, axis=1, keepdims=True)
        cy = jnp.sum(tmp[:, 128:256] * oneL, axis=1, keepdims=True)
        cz = jnp.sum(tmp[:, 256:] * oneL, axis=1, keepdims=True)
        return cx, cy, cz

    dist0, c0 = [], []
    for b in range(B):
        dist0.append(jnp.full((R, 128), 1e10, jnp.float32))
        c0.append((xs[b][0:1, 0:1], ys[b][0:1, 0:1], zs[b][0:1, 0:1]))

    def body(i, carry):
        new = []
        for b in range(B):
            dist, (cx, cy, cz) = carry[b]
            row = cx * oh0 + cy * oh1 + cz * oh2
            out_ref[b, pl.ds(i - 1, 1), :] = row
            dx = xs[b] - cx
            dy = ys[b] - cy
            dz = zs[b] - cz
            d = dx * dx + dy * dy
            d = d + dz * dz
            dist = jnp.minimum(dist, d)
            m_col = jnp.max(dist, axis=0, keepdims=True)
            r_l = jnp.min(jnp.where(dist == m_col, rowiota, R),
                          axis=0, keepdims=True)
            m = jnp.max(m_col, axis=1, keepdims=True)
            key = jnp.where(m_col == m, r_l * 128 + lane, N)
            nxt = jnp.min(key, axis=1, keepdims=True)
            new.append((dist, gather3(nxt, b)))
        return tuple(new)

    carry = lax.fori_loop(1, _NPOINT, body,
                          tuple((dist0[b], c0[b]) for b in range(B)))
    for b in range(B):
        _, (cx, cy, cz) = carry[b]
        row = cx * oh0 + cy * oh1 + cz * oh2
        out_ref[b, pl.ds(_NPOINT - 1, 1), :] = row


def _fps(xt):
    B, _, N = xt.shape
    return pl.pallas_call(
        _fps_body,
        out_shape=jax.ShapeDtypeStruct((B, _NPOINT, 128), jnp.float32),
    )(xt)


def _bq_body(xyz_ref, nt_ref, out_ref):
    N = xyz_ref.shape[1]
    W = N // 32
    x = xyz_ref[0, :, 0:1]
    y = xyz_ref[0, :, 1:2]
    z = xyz_ref[0, :, 2:3]
    cx = nt_ref[0, 0, :][None, :]
    cy = nt_ref[0, 1, :][None, :]
    cz = nt_ref[0, 2, :][None, :]
    dx = cx - x
    dy = cy - y
    dz = cz - z
    d2 = dx * dx + dy * dy
    d2 = d2 + dz * dz
    sub = lax.broadcasted_iota(jnp.int32, (N, 1), 0)
    pw = jnp.left_shift(jnp.int32(1), jnp.bitwise_and(sub, 31))
    bits = jnp.where(d2 < _RADIUS2, pw, 0)
    words = jnp.sum(bits.reshape(W, 32, _TS), axis=1)
    rowbase = lax.broadcasted_iota(jnp.int32, (W, _TS), 0) * 32
    rowi = lax.broadcasted_iota(jnp.int32, (W, _TS), 0)
    cols = []
    first = None
    for k in range(_NSAMPLE):
        b = jnp.bitwise_and(words, -words)
        e = lax.shift_right_logical(
            lax.bitcast_convert_type(b.astype(jnp.float32), jnp.int32), 23)
        bit = jnp.bitwise_and(e, 255) - 127
        cand = jnp.where(words != 0, rowbase + bit, N)
        v = jnp.min(cand, axis=0, keepdims=True)
        if k == 0:
            first = v
            cols.append(v)
        else:
            cols.append(jnp.where(v == N, first, v))
        wsel = rowi == lax.shift_right_logical(v, 5)
        pat = jnp.left_shift(jnp.int32(1), jnp.bitwise_and(v, 31))
        words = jnp.where(wsel, jnp.bitwise_xor(words, pat), words)
    out_ref[0] = jnp.concatenate(cols, axis=0)


def _ball_query(xyz, nt):
    B, N, _ = xyz.shape
    S = nt.shape[2]
    return pl.pallas_call(
        _bq_body,
        grid=(B, S // _TS),
        in_specs=[
            pl.BlockSpec((1, N, 3), lambda b, t: (b, 0, 0)),
            pl.BlockSpec((1, 3, _TS), lambda b, t: (b, 0, t)),
        ],
        out_specs=pl.BlockSpec((1, _NSAMPLE, _TS), lambda b, t: (b, 0, t)),
        out_shape=jax.ShapeDtypeStruct((B, _NSAMPLE, S), jnp.int32),
    )(xyz, nt)


def _sc_gather(tbl, idxg):
    total = idxg.shape[0]
    d = tbl.shape[1]
    per_w = total // _NW
    n_ch = per_w // _CH
    mesh = plsc.VectorSubcoreMesh(core_axis_name="c", subcore_axis_name="s")

    @functools.partial(
        pl.kernel,
        mesh=mesh,
        out_type=jax.ShapeDtypeStruct((total, d), jnp.float32),
        scratch_types=[
            pltpu.VMEM((_CH,), jnp.int32),
            pltpu.VMEM((_CH, d), jnp.float32),
            pltpu.SemaphoreType.DMA,
        ],
    )
    def gk(tbl_hbm, idx_hbm, out_hbm, idx_v, rows_v, sem):
        wid = lax.axis_index("s") * 2 + lax.axis_index("c")
        base = wid * per_w

        def step(j, carry):
            off = base + j * _CH
            pltpu.sync_copy(idx_hbm.at[pl.ds(off, _CH)], idx_v)
            pltpu.async_copy(tbl_hbm.at[idx_v], rows_v, sem).wait()
            pltpu.sync_copy(rows_v, out_hbm.at[pl.ds(off, _CH)])
            return carry

        lax.fori_loop(0, n_ch, step, 0)

    return gk(tbl, idxg)


def _mlp_body(g_ref, nx_ref, w1_ref, w1c_ref, b1_ref, w2_ref, b2_ref,
              w3_ref, b3_ref, out_ref):
    K = _NSAMPLE
    g = g_ref[0]
    h = jnp.dot(g, w1_ref[...], preferred_element_type=jnp.float32)
    c = nx_ref[0]
    ct = jnp.dot(c, w1c_ref[...], preferred_element_type=jnp.float32)
    h = h.reshape(_TS, K, h.shape[-1]) - ct[:, None, :]
    h = jnp.maximum(h + b1_ref[...][None], 0.0)
    h = h.reshape(_TS * K, h.shape[-1])
    h = jnp.maximum(
        jnp.dot(h, w2_ref[...], preferred_element_type=jnp.float32)
        + b2_ref[...], 0.0)
    h = jnp.maximum(
        jnp.dot(h, w3_ref[...], preferred_element_type=jnp.float32)
        + b3_ref[...], 0.0)
    out_ref[0] = jnp.max(h.reshape(_TS, K, h.shape[-1]), axis=1)


def _mlp(g, nxp, wbig, w1c, b1, w2, b2, w3, b3):
    B = g.shape[0]
    S = nxp.shape[1]
    D = g.shape[2]
    C3 = w3.shape[1]
    full = lambda shp: pl.BlockSpec(shp, lambda b, t: tuple(0 for _ in shp))
    return pl.pallas_call(
        _mlp_body,
        grid=(B, S // _TS),
        in_specs=[
            pl.BlockSpec((1, _TS * _NSAMPLE, D), lambda b, t: (b, t, 0)),
            pl.BlockSpec((1, _TS, 8), lambda b, t: (b, t, 0)),
            full(wbig.shape),
            full(w1c.shape),
            full(b1.shape),
            full(w2.shape),
            full(b2.shape),
            full(w3.shape),
            full(b3.shape),
        ],
        out_specs=pl.BlockSpec((1, _TS, C3), lambda b, t: (b, t, 0)),
        out_shape=jax.ShapeDtypeStruct((B, S, C3), jnp.float32),
    )(g, nxp, wbig, w1c, b1, w2, b2, w3, b3)


def kernel(xyz, features, W1, b1, W2, b2, W3, b3):
    B, N, _ = xyz.shape
    C = features.shape[1]
    S, K = _NPOINT, _NSAMPLE
    f32 = jnp.float32

    xt = jnp.transpose(xyz, (0, 2, 1))
    nx_pad = _fps(xt)
    new_xyz = nx_pad[:, :, :3]
    nt = jnp.transpose(new_xyz, (0, 2, 1))
    idx = jnp.transpose(_ball_query(xyz, nt), (0, 2, 1))

    pad = (-(C + 3)) % 128
    D = C + 3 + pad
    feats_t = jnp.transpose(features, (0, 2, 1))
    tbl = jnp.concatenate(
        [feats_t, xyz, jnp.zeros((B, N, pad), f32)], axis=-1
    ).reshape(B * N, D)
    idxg = (idx + (jnp.arange(B, dtype=jnp.int32) * N)[:, None, None]
            ).reshape(-1)
    g = _sc_gather(tbl, idxg).reshape(B, S * K, D)

    nxp = jnp.concatenate([new_xyz, jnp.zeros((B, S, 5), f32)], axis=-1)
    wbig = jnp.concatenate(
        [W1[3:], W1[:3], jnp.zeros((pad, W1.shape[1]), f32)], axis=0)
    w1c = jnp.concatenate([W1[:3], jnp.zeros((5, W1.shape[1]), f32)], axis=0)
    out = _mlp(g, nxp, wbig, w1c, b1.reshape(1, -1), W2, b2.reshape(1, -1),
               W3, b3.reshape(1, -1))
    new_features = jnp.transpose(out, (0, 2, 1))
    return (new_xyz, new_features)


# speedup vs baseline: 2.4599x; 1.0242x over previous
import functools

import jax
import jax.numpy as jnp
import numpy as np
from jax import lax
from jax.experimental import pallas as pl
from jax.experimental.pallas import tpu as pltpu
from jax.experimental.pallas import tpu_sc as plsc

_NPOINT = 512
_NSAMPLE = 32
_RADIUS2 = np.float32(0.2 ** 2)
_TS = 128
_NW = 32
_CH = 128


def _fps_body(xt_ref, out_ref):
    B = xt_ref.shape[0]
    N = xt_ref.shape[2]
    R = N // 128
    xs, ys, zs, xall = [], [], [], []
    for b in range(B):
        x = xt_ref[b, 0, :].reshape(R, 128)
        y = xt_ref[b, 1, :].reshape(R, 128)
        z = xt_ref[b, 2, :].reshape(R, 128)
        xs.append(x); ys.append(y); zs.append(z)
        xall.append(jnp.concatenate([x, y, z], axis=1))
    rowiota = lax.broadcasted_iota(jnp.int32, (R, 128), 0)
    lane = lax.broadcasted_iota(jnp.int32, (1, 128), 1)
    rowc = lax.broadcasted_iota(jnp.int32, (R, 1), 0)
    oh0 = (lane == 0).astype(jnp.float32)
    oh1 = (lane == 1).astype(jnp.float32)
    oh2 = (lane == 2).astype(jnp.float32)

    def gather3(nxt, b):
        r = lax.shift_right_logical(nxt, 7)
        l = jnp.bitwise_and(nxt, 127)
        oneR = (rowc == r).astype(jnp.float32)
        oneL = (lane == l).astype(jnp.float32)
        tmp = jnp.sum(xall[b] * oneR, axis=0, keepdims=True)
        cx = jnp.sum(tmp[:, :128] * oneL, axis=1, keepdims=True)
        cy = jnp.sum(tmp[:, 128:256] * oneL, axis=1, keepdims=True)
        cz = jnp.sum(tmp[:, 256:] * oneL, axis=1, keepdims=True)
        return cx, cy, cz

    dist0, c0 = [], []
    for b in range(B):
        dist0.append(jnp.full((R, 128), 1e10, jnp.float32))
        c0.append((xs[b][0:1, 0:1], ys[b][0:1, 0:1], zs[b][0:1, 0:1]))

    def body(i, carry):
        new = []
        for b in range(B):
            dist, (cx, cy, cz) = carry[b]
            row = cx * oh0 + cy * oh1 + cz * oh2
            out_ref[b, pl.ds(i - 1, 1), :] = row
            dx = xs[b] - cx
            dy = ys[b] - cy
            dz = zs[b] - cz
            d = dx * dx + dy * dy
            d = d + dz * dz
            dist = jnp.minimum(dist, d)
            m_col = jnp.max(dist, axis=0, keepdims=True)
            r_l = jnp.min(jnp.where(dist == m_col, rowiota, R),
                          axis=0, keepdims=True)
            m = jnp.max(m_col, axis=1, keepdims=True)
            key = jnp.where(m_col == m, r_l * 128 + lane, N)
            nxt = jnp.min(key, axis=1, keepdims=True)
            new.append((dist, gather3(nxt, b)))
        return tuple(new)

    def body2(j, carry):
        return body(2 * j + 2, body(2 * j + 1, carry))

    carry = lax.fori_loop(0, (_NPOINT - 2) // 2, body2,
                          tuple((dist0[b], c0[b]) for b in range(B)))
    carry = body(_NPOINT - 1, carry)
    for b in range(B):
        _, (cx, cy, cz) = carry[b]
        row = cx * oh0 + cy * oh1 + cz * oh2
        out_ref[b, pl.ds(_NPOINT - 1, 1), :] = row


def _fps(xt):
    B, _, N = xt.shape
    return pl.pallas_call(
        _fps_body,
        out_shape=jax.ShapeDtypeStruct((B, _NPOINT, 128), jnp.float32),
    )(xt)


def _bq_body(xyz_ref, nt_ref, out_ref):
    N = xyz_ref.shape[1]
    W = N // 32
    x = xyz_ref[0, :, 0:1]
    y = xyz_ref[0, :, 1:2]
    z = xyz_ref[0, :, 2:3]
    cx = nt_ref[0, 0, :][None, :]
    cy = nt_ref[0, 1, :][None, :]
    cz = nt_ref[0, 2, :][None, :]
    dx = cx - x
    dy = cy - y
    dz = cz - z
    d2 = dx * dx + dy * dy
    d2 = d2 + dz * dz
    sub = lax.broadcasted_iota(jnp.int32, (N, 1), 0)
    pw = jnp.left_shift(jnp.int32(1), jnp.bitwise_and(sub, 31))
    bits = jnp.where(d2 < _RADIUS2, pw, 0)
    words = jnp.sum(bits.reshape(W, 32, _TS), axis=1)
    rowbase = lax.broadcasted_iota(jnp.int32, (W, _TS), 0) * 32
    rowi = lax.broadcasted_iota(jnp.int32, (W, _TS), 0)
    cols = []
    first = None
    for k in range(_NSAMPLE):
        b = jnp.bitwise_and(words, -words)
        e = lax.shift_right_logical(
            lax.bitcast_convert_type(b.astype(jnp.float32), jnp.int32), 23)
        bit = jnp.bitwise_and(e, 255) - 127
        cand = jnp.where(words != 0, rowbase + bit, N)
        v = jnp.min(cand, axis=0, keepdims=True)
        if k == 0:
            first = v
            cols.append(v)
        else:
            cols.append(jnp.where(v == N, first, v))
        wsel = rowi == lax.shift_right_logical(v, 5)
        pat = jnp.left_shift(jnp.int32(1), jnp.bitwise_and(v, 31))
        words = jnp.where(wsel, jnp.bitwise_xor(words, pat), words)
    out_ref[0] = jnp.concatenate(cols, axis=0)


def _ball_query(xyz, nt):
    B, N, _ = xyz.shape
    S = nt.shape[2]
    return pl.pallas_call(
        _bq_body,
        grid=(B, S // _TS),
        in_specs=[
            pl.BlockSpec((1, N, 3), lambda b, t: (b, 0, 0)),
            pl.BlockSpec((1, 3, _TS), lambda b, t: (b, 0, t)),
        ],
        out_specs=pl.BlockSpec((1, _NSAMPLE, _TS), lambda b, t: (b, 0, t)),
        out_shape=jax.ShapeDtypeStruct((B, _NSAMPLE, S), jnp.int32),
    )(xyz, nt)


def _sc_gather(tbl, idxg):
    total = idxg.shape[0]
    d = tbl.shape[1]
    per_w = total // _NW
    n_ch = per_w // _CH
    mesh = plsc.VectorSubcoreMesh(core_axis_name="c", subcore_axis_name="s")

    @functools.partial(
        pl.kernel,
        mesh=mesh,
        out_type=jax.ShapeDtypeStruct((total, d), jnp.float32),
        scratch_types=[
            pltpu.VMEM((_CH,), jnp.int32),
            pltpu.VMEM((_CH, d), jnp.float32),
            pltpu.SemaphoreType.DMA,
        ],
    )
    def gk(tbl_hbm, idx_hbm, out_hbm, idx_v, rows_v, sem):
        wid = lax.axis_index("s") * 2 + lax.axis_index("c")
        base = wid * per_w

        def step(j, carry):
            off = base + j * _CH
            pltpu.sync_copy(idx_hbm.at[pl.ds(off, _CH)], idx_v)
            pltpu.async_copy(tbl_hbm.at[idx_v], rows_v, sem).wait()
            pltpu.sync_copy(rows_v, out_hbm.at[pl.ds(off, _CH)])
            return carry

        lax.fori_loop(0, n_ch, step, 0)

    return gk(tbl, idxg)


def _mlp_body(g_ref, nx_ref, w1_ref, w1c_ref, b1_ref, w2_ref, b2_ref,
              w3_ref, b3_ref, out_ref):
    K = _NSAMPLE
    g = g_ref[0]
    h = jnp.dot(g, w1_ref[...], preferred_element_type=jnp.float32)
    c = nx_ref[0]
    ct = jnp.dot(c, w1c_ref[...], preferred_element_type=jnp.float32)
    h = h.reshape(_TS, K, h.shape[-1]) - ct[:, None, :]
    h = jnp.maximum(h + b1_ref[...][None], 0.0)
    h = h.reshape(_TS * K, h.shape[-1])
    h = jnp.maximum(
        jnp.dot(h, w2_ref[...], preferred_element_type=jnp.float32)
        + b2_ref[...], 0.0)
    h = jnp.maximum(
        jnp.dot(h, w3_ref[...], preferred_element_type=jnp.float32)
        + b3_ref[...], 0.0)
    out_ref[0] = jnp.max(h.reshape(_TS, K, h.shape[-1]), axis=1)


def _mlp(g, nxp, wbig, w1c, b1, w2, b2, w3, b3):
    B = g.shape[0]
    S = nxp.shape[1]
    D = g.shape[2]
    C3 = w3.shape[1]
    full = lambda shp: pl.BlockSpec(shp, lambda b, t: tuple(0 for _ in shp))
    return pl.pallas_call(
        _mlp_body,
        grid=(B, S // _TS),
        in_specs=[
            pl.BlockSpec((1, _TS * _NSAMPLE, D), lambda b, t: (b, t, 0)),
            pl.BlockSpec((1, _TS, 8), lambda b, t: (b, t, 0)),
            full(wbig.shape),
            full(w1c.shape),
            full(b1.shape),
            full(w2.shape),
            full(b2.shape),
            full(w3.shape),
            full(b3.shape),
        ],
        out_specs=pl.BlockSpec((1, _TS, C3), lambda b, t: (b, t, 0)),
        out_shape=jax.ShapeDtypeStruct((B, S, C3), jnp.float32),
    )(g, nxp, wbig, w1c, b1, w2, b2, w3, b3)


def kernel(xyz, features, W1, b1, W2, b2, W3, b3):
    B, N, _ = xyz.shape
    C = features.shape[1]
    S, K = _NPOINT, _NSAMPLE
    f32 = jnp.float32

    xt = jnp.transpose(xyz, (0, 2, 1))
    nx_pad = _fps(xt)
    new_xyz = nx_pad[:, :, :3]
    nt = jnp.transpose(new_xyz, (0, 2, 1))
    idx = jnp.transpose(_ball_query(xyz, nt), (0, 2, 1))

    pad = (-(C + 3)) % 128
    D = C + 3 + pad
    feats_t = jnp.transpose(features, (0, 2, 1))
    tbl = jnp.concatenate(
        [feats_t, xyz, jnp.zeros((B, N, pad), f32)], axis=-1
    ).reshape(B * N, D)
    idxg = (idx + (jnp.arange(B, dtype=jnp.int32) * N)[:, None, None]
            ).reshape(-1)
    g = _sc_gather(tbl, idxg).reshape(B, S * K, D)

    nxp = jnp.concatenate([new_xyz, jnp.zeros((B, S, 5), f32)], axis=-1)
    wbig = jnp.concatenate(
        [W1[3:], W1[:3], jnp.zeros((pad, W1.shape[1]), f32)], axis=0)
    w1c = jnp.concatenate([W1[:3], jnp.zeros((5, W1.shape[1]), f32)], axis=0)
    out = _mlp(g, nxp, wbig, w1c, b1.reshape(1, -1), W2, b2.reshape(1, -1),
               W3, b3.reshape(1, -1))
    new_features = jnp.transpose(out, (0, 2, 1))
    return (new_xyz, new_features)


# speedup vs baseline: 2.5298x; 1.0284x over previous
import functools

import jax
import jax.numpy as jnp
import numpy as np
from jax import lax
from jax.experimental import pallas as pl
from jax.experimental.pallas import tpu as pltpu
from jax.experimental.pallas import tpu_sc as plsc

_NPOINT = 512
_NSAMPLE = 32
_RADIUS2 = np.float32(0.2 ** 2)
_TS = 256
_NW = 32
_CH = 128


def _fps_body(xt_ref, out_ref):
    B = xt_ref.shape[0]
    N = xt_ref.shape[2]
    R = N // 128
    xs, ys, zs, xall = [], [], [], []
    for b in range(B):
        x = xt_ref[b, 0, :].reshape(R, 128)
        y = xt_ref[b, 1, :].reshape(R, 128)
        z = xt_ref[b, 2, :].reshape(R, 128)
        xs.append(x); ys.append(y); zs.append(z)
        xall.append(jnp.concatenate([x, y, z], axis=1))
    rowiota = lax.broadcasted_iota(jnp.int32, (R, 128), 0)
    lane = lax.broadcasted_iota(jnp.int32, (1, 128), 1)
    rowc = lax.broadcasted_iota(jnp.int32, (R, 1), 0)
    oh0 = (lane == 0).astype(jnp.float32)
    oh1 = (lane == 1).astype(jnp.float32)
    oh2 = (lane == 2).astype(jnp.float32)

    def gather3(nxt, b):
        r = lax.shift_right_logical(nxt, 7)
        l = jnp.bitwise_and(nxt, 127)
        oneR = (rowc == r).astype(jnp.float32)
        oneL = (lane == l).astype(jnp.float32)
        tmp = jnp.sum(xall[b] * oneR, axis=0, keepdims=True)
        cx = jnp.sum(tmp[:, :128] * oneL, axis=1, keepdims=True)
        cy = jnp.sum(tmp[:, 128:256] * oneL, axis=1, keepdims=True)
        cz = jnp.sum(tmp[:, 256:] * oneL, axis=1, keepdims=True)
        return cx, cy, cz

    dist0, c0 = [], []
    for b in range(B):
        dist0.append(jnp.full((R, 128), 1e10, jnp.float32))
        c0.append((xs[b][0:1, 0:1], ys[b][0:1, 0:1], zs[b][0:1, 0:1]))

    def body(i, carry):
        new = []
        for b in range(B):
            dist, (cx, cy, cz) = carry[b]
            row = cx * oh0 + cy * oh1 + cz * oh2
            out_ref[b, pl.ds(i - 1, 1), :] = row
            dx = xs[b] - cx
            dy = ys[b] - cy
            dz = zs[b] - cz
            d = dx * dx + dy * dy
            d = d + dz * dz
            dist = jnp.minimum(dist, d)
            m_col = jnp.max(dist, axis=0, keepdims=True)
            r_l = jnp.min(jnp.where(dist == m_col, rowiota, R),
                          axis=0, keepdims=True)
            m = jnp.max(m_col, axis=1, keepdims=True)
            key = jnp.where(m_col == m, r_l * 128 + lane, N)
            nxt = jnp.min(key, axis=1, keepdims=True)
            new.append((dist, gather3(nxt, b)))
        return tuple(new)

    carry = lax.fori_loop(1, _NPOINT, body,
                          tuple((dist0[b], c0[b]) for b in range(B)))
    for b in range(B):
        _, (cx, cy, cz) = carry[b]
        row = cx * oh0 + cy * oh1 + cz * oh2
        out_ref[b, pl.ds(_NPOINT - 1, 1), :] = row


def _fps(xt):
    B, _, N = xt.shape
    return pl.pallas_call(
        _fps_body,
        out_shape=jax.ShapeDtypeStruct((B, _NPOINT, 128), jnp.float32),
    )(xt)


def _bq_body(xyz_ref, nt_ref, out_ref):
    N = xyz_ref.shape[1]
    W = N // 32
    x = xyz_ref[0, :, 0:1]
    y = xyz_ref[0, :, 1:2]
    z = xyz_ref[0, :, 2:3]
    cx = nt_ref[0, 0, :][None, :]
    cy = nt_ref[0, 1, :][None, :]
    cz = nt_ref[0, 2, :][None, :]
    dx = cx - x
    dy = cy - y
    dz = cz - z
    d2 = dx * dx + dy * dy
    d2 = d2 + dz * dz
    sub = lax.broadcasted_iota(jnp.int32, (N, 1), 0)
    pw = jnp.left_shift(jnp.int32(1), jnp.bitwise_and(sub, 31))
    bits = jnp.where(d2 < _RADIUS2, pw, 0)
    words = jnp.sum(bits.reshape(W, 32, _TS), axis=1)
    rowbase = lax.broadcasted_iota(jnp.int32, (W, _TS), 0) * 32
    rowi = lax.broadcasted_iota(jnp.int32, (W, _TS), 0)
    cols = []
    first = None
    for k in range(_NSAMPLE):
        b = jnp.bitwise_and(words, -words)
        e = lax.shift_right_logical(
            lax.bitcast_convert_type(b.astype(jnp.float32), jnp.int32), 23)
        bit = jnp.bitwise_and(e, 255) - 127
        cand = jnp.where(words != 0, rowbase + bit, N)
        v = jnp.min(cand, axis=0, keepdims=True)
        if k == 0:
            first = v
            cols.append(v)
        else:
            cols.append(jnp.where(v == N, first, v))
        wsel = rowi == lax.shift_right_logical(v, 5)
        pat = jnp.left_shift(jnp.int32(1), jnp.bitwise_and(v, 31))
        words = jnp.where(wsel, jnp.bitwise_xor(words, pat), words)
    out_ref[0] = jnp.concatenate(cols, axis=0)


def _ball_query(xyz, nt):
    B, N, _ = xyz.shape
    S = nt.shape[2]
    return pl.pallas_call(
        _bq_body,
        grid=(B, S // _TS),
        in_specs=[
            pl.BlockSpec((1, N, 3), lambda b, t: (b, 0, 0)),
            pl.BlockSpec((1, 3, _TS), lambda b, t: (b, 0, t)),
        ],
        out_specs=pl.BlockSpec((1, _NSAMPLE, _TS), lambda b, t: (b, 0, t)),
        out_shape=jax.ShapeDtypeStruct((B, _NSAMPLE, S), jnp.int32),
    )(xyz, nt)


def _sc_gather(tbl, idxg):
    total = idxg.shape[0]
    d = tbl.shape[1]
    per_w = total // _NW
    n_ch = per_w // _CH
    mesh = plsc.VectorSubcoreMesh(core_axis_name="c", subcore_axis_name="s")

    @functools.partial(
        pl.kernel,
        mesh=mesh,
        out_type=jax.ShapeDtypeStruct((total, d), jnp.float32),
        scratch_types=[
            pltpu.VMEM((_CH,), jnp.int32),
            pltpu.VMEM((_CH, d), jnp.float32),
            pltpu.SemaphoreType.DMA,
        ],
    )
    def gk(tbl_hbm, idx_hbm, out_hbm, idx_v, rows_v, sem):
        wid = lax.axis_index("s") * 2 + lax.axis_index("c")
        base = wid * per_w

        def step(j, carry):
            off = base + j * _CH
            pltpu.sync_copy(idx_hbm.at[pl.ds(off, _CH)], idx_v)
            pltpu.async_copy(tbl_hbm.at[idx_v], rows_v, sem).wait()
            pltpu.sync_copy(rows_v, out_hbm.at[pl.ds(off, _CH)])
            return carry

        lax.fori_loop(0, n_ch, step, 0)

    return gk(tbl, idxg)


def _mlp_body(g_ref, nx_ref, w1_ref, w1c_ref, b1_ref, w2_ref, b2_ref,
              w3_ref, b3_ref, out_ref):
    K = _NSAMPLE
    g = g_ref[0]
    h = jnp.dot(g, w1_ref[...], preferred_element_type=jnp.float32)
    c = nx_ref[0]
    ct = jnp.dot(c, w1c_ref[...], preferred_element_type=jnp.float32)
    h = h.reshape(_TS, K, h.shape[-1]) - ct[:, None, :]
    h = jnp.maximum(h + b1_ref[...][None], 0.0)
    h = h.reshape(_TS * K, h.shape[-1])
    h = jnp.maximum(
        jnp.dot(h, w2_ref[...], preferred_element_type=jnp.float32)
        + b2_ref[...], 0.0)
    h = jnp.maximum(
        jnp.dot(h, w3_ref[...], preferred_element_type=jnp.float32)
        + b3_ref[...], 0.0)
    out_ref[0] = jnp.max(h.reshape(_TS, K, h.shape[-1]), axis=1)


def _mlp(g, nxp, wbig, w1c, b1, w2, b2, w3, b3):
    B = g.shape[0]
    S = nxp.shape[1]
    D = g.shape[2]
    C3 = w3.shape[1]
    full = lambda shp: pl.BlockSpec(shp, lambda b, t: tuple(0 for _ in shp))
    return pl.pallas_call(
        _mlp_body,
        grid=(B, S // _TS),
        in_specs=[
            pl.BlockSpec((1, _TS * _NSAMPLE, D), lambda b, t: (b, t, 0)),
            pl.BlockSpec((1, _TS, 8), lambda b, t: (b, t, 0)),
            full(wbig.shape),
            full(w1c.shape),
            full(b1.shape),
            full(w2.shape),
            full(b2.shape),
            full(w3.shape),
            full(b3.shape),
        ],
        out_specs=pl.BlockSpec((1, _TS, C3), lambda b, t: (b, t, 0)),
        out_shape=jax.ShapeDtypeStruct((B, S, C3), jnp.float32),
    )(g, nxp, wbig, w1c, b1, w2, b2, w3, b3)


def kernel(xyz, features, W1, b1, W2, b2, W3, b3):
    B, N, _ = xyz.shape
    C = features.shape[1]
    S, K = _NPOINT, _NSAMPLE
    f32 = jnp.float32

    xt = jnp.transpose(xyz, (0, 2, 1))
    nx_pad = _fps(xt)
    new_xyz = nx_pad[:, :, :3]
    nt = jnp.transpose(new_xyz, (0, 2, 1))
    idx = jnp.transpose(_ball_query(xyz, nt), (0, 2, 1))

    pad = (-(C + 3)) % 128
    D = C + 3 + pad
    feats_t = jnp.transpose(features, (0, 2, 1))
    tbl = jnp.concatenate(
        [feats_t, xyz, jnp.zeros((B, N, pad), f32)], axis=-1
    ).reshape(B * N, D)
    idxg = (idx + (jnp.arange(B, dtype=jnp.int32) * N)[:, None, None]
            ).reshape(-1)
    g = _sc_gather(tbl, idxg).reshape(B, S * K, D)

    nxp = jnp.concatenate([new_xyz, jnp.zeros((B, S, 5), f32)], axis=-1)
    wbig = jnp.concatenate(
        [W1[3:], W1[:3], jnp.zeros((pad, W1.shape[1]), f32)], axis=0)
    w1c = jnp.concatenate([W1[:3], jnp.zeros((5, W1.shape[1]), f32)], axis=0)
    out = _mlp(g, nxp, wbig, w1c, b1.reshape(1, -1), W2, b2.reshape(1, -1),
               W3, b3.reshape(1, -1))
    new_features = jnp.transpose(out, (0, 2, 1))
    return (new_xyz, new_features)


# speedup vs baseline: 2.5656x; 1.0141x over previous
import functools

import jax
import jax.numpy as jnp
import numpy as np
from jax import lax
from jax.experimental import pallas as pl
from jax.experimental.pallas import tpu as pltpu
from jax.experimental.pallas import tpu_sc as plsc

_NPOINT = 512
_NSAMPLE = 32
_RADIUS2 = np.float32(0.2 ** 2)
_TS = 512
_NW = 32
_CH = 128


def _fps_body(xt_ref, out_ref):
    B = xt_ref.shape[0]
    N = xt_ref.shape[2]
    R = N // 128
    xs, ys, zs, xall = [], [], [], []
    for b in range(B):
        x = xt_ref[b, 0, :].reshape(R, 128)
        y = xt_ref[b, 1, :].reshape(R, 128)
        z = xt_ref[b, 2, :].reshape(R, 128)
        xs.append(x); ys.append(y); zs.append(z)
        xall.append(jnp.concatenate([x, y, z], axis=1))
    rowiota = lax.broadcasted_iota(jnp.int32, (R, 128), 0)
    lane = lax.broadcasted_iota(jnp.int32, (1, 128), 1)
    rowc = lax.broadcasted_iota(jnp.int32, (R, 1), 0)
    oh0 = (lane == 0).astype(jnp.float32)
    oh1 = (lane == 1).astype(jnp.float32)
    oh2 = (lane == 2).astype(jnp.float32)

    def gather3(nxt, b):
        r = lax.shift_right_logical(nxt, 7)
        l = jnp.bitwise_and(nxt, 127)
        oneR = (rowc == r).astype(jnp.float32)
        oneL = (lane == l).astype(jnp.float32)
        tmp = jnp.sum(xall[b] * oneR, axis=0, keepdims=True)
        cx = jnp.sum(tmp[:, :128] * oneL, axis=1, keepdims=True)
        cy = jnp.sum(tmp[:, 128:256] * oneL, axis=1, keepdims=True)
        cz = jnp.sum(tmp[:, 256:] * oneL, axis=1, keepdims=True)
        return cx, cy, cz

    dist0, c0 = [], []
    for b in range(B):
        dist0.append(jnp.full((R, 128), 1e10, jnp.float32))
        c0.append((xs[b][0:1, 0:1], ys[b][0:1, 0:1], zs[b][0:1, 0:1]))

    def body(i, carry):
        new = []
        for b in range(B):
            dist, (cx, cy, cz) = carry[b]
            row = cx * oh0 + cy * oh1 + cz * oh2
            out_ref[b, pl.ds(i - 1, 1), :] = row
            dx = xs[b] - cx
            dy = ys[b] - cy
            dz = zs[b] - cz
            d = dx * dx + dy * dy
            d = d + dz * dz
            dist = jnp.minimum(dist, d)
            m_col = jnp.max(dist, axis=0, keepdims=True)
            r_l = jnp.min(jnp.where(dist == m_col, rowiota, R),
                          axis=0, keepdims=True)
            m = jnp.max(m_col, axis=1, keepdims=True)
            key = jnp.where(m_col == m, r_l * 128 + lane, N)
            nxt = jnp.min(key, axis=1, keepdims=True)
            new.append((dist, gather3(nxt, b)))
        return tuple(new)

    carry = lax.fori_loop(1, _NPOINT, body,
                          tuple((dist0[b], c0[b]) for b in range(B)))
    for b in range(B):
        _, (cx, cy, cz) = carry[b]
        row = cx * oh0 + cy * oh1 + cz * oh2
        out_ref[b, pl.ds(_NPOINT - 1, 1), :] = row


def _fps(xt):
    B, _, N = xt.shape
    return pl.pallas_call(
        _fps_body,
        out_shape=jax.ShapeDtypeStruct((B, _NPOINT, 128), jnp.float32),
    )(xt)


def _bq_body(xyz_ref, nt_ref, out_ref):
    N = xyz_ref.shape[1]
    W = N // 32
    x = xyz_ref[0, :, 0:1]
    y = xyz_ref[0, :, 1:2]
    z = xyz_ref[0, :, 2:3]
    cx = nt_ref[0, 0, :][None, :]
    cy = nt_ref[0, 1, :][None, :]
    cz = nt_ref[0, 2, :][None, :]
    dx = cx - x
    dy = cy - y
    dz = cz - z
    d2 = dx * dx + dy * dy
    d2 = d2 + dz * dz
    sub = lax.broadcasted_iota(jnp.int32, (N, 1), 0)
    pw = jnp.left_shift(jnp.int32(1), jnp.bitwise_and(sub, 31))
    bits = jnp.where(d2 < _RADIUS2, pw, 0)
    words = jnp.sum(bits.reshape(W, 32, _TS), axis=1)
    rowbase = lax.broadcasted_iota(jnp.int32, (W, _TS), 0) * 32
    rowi = lax.broadcasted_iota(jnp.int32, (W, _TS), 0)
    cols = []
    first = None
    for k in range(_NSAMPLE):
        b = jnp.bitwise_and(words, -words)
        e = lax.shift_right_logical(
            lax.bitcast_convert_type(b.astype(jnp.float32), jnp.int32), 23)
        bit = jnp.bitwise_and(e, 255) - 127
        cand = jnp.where(words != 0, rowbase + bit, N)
        v = jnp.min(cand, axis=0, keepdims=True)
        if k == 0:
            first = v
            cols.append(v)
        else:
            cols.append(jnp.where(v == N, first, v))
        wsel = rowi == lax.shift_right_logical(v, 5)
        pat = jnp.left_shift(jnp.int32(1), jnp.bitwise_and(v, 31))
        words = jnp.where(wsel, jnp.bitwise_xor(words, pat), words)
    out_ref[0] = jnp.concatenate(cols, axis=0)


def _ball_query(xyz, nt):
    B, N, _ = xyz.shape
    S = nt.shape[2]
    return pl.pallas_call(
        _bq_body,
        grid=(B, S // _TS),
        in_specs=[
            pl.BlockSpec((1, N, 3), lambda b, t: (b, 0, 0)),
            pl.BlockSpec((1, 3, _TS), lambda b, t: (b, 0, t)),
        ],
        out_specs=pl.BlockSpec((1, _NSAMPLE, _TS), lambda b, t: (b, 0, t)),
        out_shape=jax.ShapeDtypeStruct((B, _NSAMPLE, S), jnp.int32),
    )(xyz, nt)


def _sc_gather(tbl, idxg):
    total = idxg.shape[0]
    d = tbl.shape[1]
    per_w = total // _NW
    n_ch = per_w // _CH
    mesh = plsc.VectorSubcoreMesh(core_axis_name="c", subcore_axis_name="s")

    @functools.partial(
        pl.kernel,
        mesh=mesh,
        out_type=jax.ShapeDtypeStruct((total, d), jnp.float32),
        scratch_types=[
            pltpu.VMEM((_CH,), jnp.int32),
            pltpu.VMEM((_CH, d), jnp.float32),
            pltpu.SemaphoreType.DMA,
        ],
    )
    def gk(tbl_hbm, idx_hbm, out_hbm, idx_v, rows_v, sem):
        wid = lax.axis_index("s") * 2 + lax.axis_index("c")
        base = wid * per_w

        def step(j, carry):
            off = base + j * _CH
            pltpu.sync_copy(idx_hbm.at[pl.ds(off, _CH)], idx_v)
            pltpu.async_copy(tbl_hbm.at[idx_v], rows_v, sem).wait()
            pltpu.sync_copy(rows_v, out_hbm.at[pl.ds(off, _CH)])
            return carry

        lax.fori_loop(0, n_ch, step, 0)

    return gk(tbl, idxg)


def _mlp_body(g_ref, nx_ref, w1_ref, w1c_ref, b1_ref, w2_ref, b2_ref,
              w3_ref, b3_ref, out_ref):
    K = _NSAMPLE
    g = g_ref[0]
    h = jnp.dot(g, w1_ref[...], preferred_element_type=jnp.float32)
    c = nx_ref[0]
    ct = jnp.dot(c, w1c_ref[...], preferred_element_type=jnp.float32)
    h = h.reshape(_TS, K, h.shape[-1]) - ct[:, None, :]
    h = jnp.maximum(h + b1_ref[...][None], 0.0)
    h = h.reshape(_TS * K, h.shape[-1])
    h = jnp.maximum(
        jnp.dot(h, w2_ref[...], preferred_element_type=jnp.float32)
        + b2_ref[...], 0.0)
    h = jnp.maximum(
        jnp.dot(h, w3_ref[...], preferred_element_type=jnp.float32)
        + b3_ref[...], 0.0)
    out_ref[0] = jnp.max(h.reshape(_TS, K, h.shape[-1]), axis=1)


def _mlp(g, nxp, wbig, w1c, b1, w2, b2, w3, b3):
    B = g.shape[0]
    S = nxp.shape[1]
    D = g.shape[2]
    C3 = w3.shape[1]
    full = lambda shp: pl.BlockSpec(shp, lambda b, t: tuple(0 for _ in shp))
    return pl.pallas_call(
        _mlp_body,
        grid=(B, S // _TS),
        in_specs=[
            pl.BlockSpec((1, _TS * _NSAMPLE, D), lambda b, t: (b, t, 0)),
            pl.BlockSpec((1, _TS, 8), lambda b, t: (b, t, 0)),
            full(wbig.shape),
            full(w1c.shape),
            full(b1.shape),
            full(w2.shape),
            full(b2.shape),
            full(w3.shape),
            full(b3.shape),
        ],
        out_specs=pl.BlockSpec((1, _TS, C3), lambda b, t: (b, t, 0)),
        out_shape=jax.ShapeDtypeStruct((B, S, C3), jnp.float32),
    )(g, nxp, wbig, w1c, b1, w2, b2, w3, b3)


def kernel(xyz, features, W1, b1, W2, b2, W3, b3):
    B, N, _ = xyz.shape
    C = features.shape[1]
    S, K = _NPOINT, _NSAMPLE
    f32 = jnp.float32

    xt = jnp.transpose(xyz, (0, 2, 1))
    nx_pad = _fps(xt)
    new_xyz = nx_pad[:, :, :3]
    nt = jnp.transpose(new_xyz, (0, 2, 1))
    idx = jnp.transpose(_ball_query(xyz, nt), (0, 2, 1))

    pad = (-(C + 3)) % 128
    D = C + 3 + pad
    feats_t = jnp.transpose(features, (0, 2, 1))
    tbl = jnp.concatenate(
        [feats_t, xyz, jnp.zeros((B, N, pad), f32)], axis=-1
    ).reshape(B * N, D)
    idxg = (idx + (jnp.arange(B, dtype=jnp.int32) * N)[:, None, None]
            ).reshape(-1)
    g = _sc_gather(tbl, idxg).reshape(B, S * K, D)

    nxp = jnp.concatenate([new_xyz, jnp.zeros((B, S, 5), f32)], axis=-1)
    wbig = jnp.concatenate(
        [W1[3:], W1[:3], jnp.zeros((pad, W1.shape[1]), f32)], axis=0)
    w1c = jnp.concatenate([W1[:3], jnp.zeros((5, W1.shape[1]), f32)], axis=0)
    out = _mlp(g, nxp, wbig, w1c, b1.reshape(1, -1), W2, b2.reshape(1, -1),
               W3, b3.reshape(1, -1))
    new_features = jnp.transpose(out, (0, 2, 1))
    return (new_xyz, new_features)
